# baseline (speedup 1.0000x reference)
import jax
import jax.numpy as jnp
from jax.experimental import pallas as pl

N_AST = 100000
N_BLK = 10000
HID = 128
HEADS = 4
DH = HID // HEADS


def _gat(src, dst, etype, x, W, a_s, a_d, et_bias):
    n = x.shape[0]
    h = (x @ W).reshape(n, HEADS, DH)
    alpha_src = jnp.sum(h * a_s[None, :, :], axis=-1)
    alpha_dst = jnp.sum(h * a_d[None, :, :], axis=-1)
    e = jax.nn.leaky_relu(alpha_src[src] + alpha_dst[dst] + et_bias[etype], 0.2)
    m = jax.ops.segment_max(e, dst, num_segments=n)
    m = jnp.where(jnp.isfinite(m), m, 0.0)
    ex = jnp.exp(e - m[dst])
    denom = jax.ops.segment_sum(ex, dst, num_segments=n)
    attn = ex / (denom[dst] + 1e-9)
    out = jax.ops.segment_sum(h[src] * attn[:, :, None], dst, num_segments=n)
    return out.reshape(n, HID)


def _tail_kernel(fea_ref, wmlp_ref, bmlp_ref, label_ref, sig_ref, loss_ref, pooled_ref):
    pooled = jnp.max(fea_ref[...], axis=0, keepdims=True)
    logits = pooled @ wmlp_ref[...] + bmlp_ref[...]
    pooled_ref[...] = pooled
    sig_ref[...] = jax.nn.sigmoid(logits)
    l = logits.reshape(1)
    y = label_ref[...]
    loss = jnp.maximum(l, 0.0) - l * y + jnp.log1p(jnp.exp(-jnp.abs(l)))
    loss_ref[...] = jnp.mean(loss).reshape(1)


def kernel(ast_tokens, ast_parent, emb_block_id, cfg_edge_index, cfg_edge_type, label,
           embed_table, W_self, W_child, b_conv, W_block, b_block,
           W_gi, a_src_gi, a_dst_gi, et_gi,
           W_go, a_src_go, a_dst_go, et_go,
           W_mlp, b_mlp):
    x = embed_table[ast_tokens]
    child_agg = jax.ops.segment_sum(x, ast_parent, num_segments=N_AST)
    conv_out = jax.nn.relu(x @ W_self + child_agg @ W_child + b_conv)
    block_sum = jax.ops.segment_sum(conv_out, emb_block_id, num_segments=N_BLK)
    block_features = jax.nn.relu(block_sum @ W_block + b_block)
    src = cfg_edge_index[0]
    dst = cfg_edge_index[1]
    gat_in = _gat(src, dst, cfg_edge_type, block_features, W_gi, a_src_gi, a_dst_gi, et_gi)
    gat_out = _gat(dst, src, cfg_edge_type, block_features, W_go, a_src_go, a_dst_go, et_go)
    fea = gat_in + gat_out + block_features

    sig, loss, pooled = pl.pallas_call(
        _tail_kernel,
        out_shape=(
            jax.ShapeDtypeStruct((1, 1), jnp.float32),
            jax.ShapeDtypeStruct((1,), jnp.float32),
            jax.ShapeDtypeStruct((1, HID), jnp.float32),
        ),
    )(fea, W_mlp, b_mlp, label)
    return (sig, loss.reshape(()), pooled)


# speedup vs baseline: 23.1443x; 23.1443x over previous
import functools

import jax
import jax.numpy as jnp
from jax import lax
from jax.experimental import pallas as pl
from jax.experimental.pallas import tpu as pltpu
from jax.experimental.pallas import tpu_sc as plsc

N_AST = 100000
N_BLK = 10000
E_CFG = 320000
TOKEN_SIZE = 1000
FEAT = 128
HID = 128
HEADS = 4
DH = HID // HEADS

PAD_N = 102400
N_CHUNK = 25
CH = 4096
SC0_CHUNKS = 13
CHUNK_ROWS = CH + 8
BLK_ROWS = 10112
TPT = PAD_N // 16
SCAN_IT = TPT // 16
MB = 128
NPT = PAD_N // 32
NBAT = NPT // 128
EPT = E_CFG // 16
EB = 80
NEB = EPT // EB
AUGW = 128
DUMPP = CH * 1024
PREC = lax.Precision.HIGHEST


def _mesh():
    return plsc.VectorSubcoreMesh(core_axis_name="c", subcore_axis_name="s",
                                  num_cores=2, num_subcores=16)


_SC_PARAMS = pltpu.CompilerParams(needs_layout_passes=False)


def _prep_body(emb_ref, ws_ref, wc_ref, ts_ref, tch_ref):
    ts_ref[...] = jnp.dot(emb_ref[...], ws_ref[...], precision=PREC)
    tch_ref[...] = jnp.dot(emb_ref[...], wc_ref[...], precision=PREC)


def _prep(embed_table, w_self, w_child):
    return pl.pallas_call(
        _prep_body,
        out_shape=(
            jax.ShapeDtypeStruct((TOKEN_SIZE, HID), jnp.float32),
            jax.ShapeDtypeStruct((TOKEN_SIZE, HID), jnp.float32),
        ),
    )(embed_table, w_self, w_child)


def _scatter_body(tok_hbm, par_hbm, tch_hbm, c2_hbm,
                  par_res, tok_res, mpacked, pidx_stage, tok_stage, rows,
                  chunk_acc, sem):
    c = lax.axis_index("c")
    s = lax.axis_index("s")
    pltpu.sync_copy(par_hbm.at[pl.ds(s * TPT, TPT)], par_res)
    pltpu.sync_copy(tok_hbm.at[pl.ds(s * TPT, TPT)], tok_res)

    zeros16 = jnp.zeros((16,), jnp.float32)
    dumpv = jnp.full((16,), DUMPP, jnp.int32)
    lane16 = lax.iota(jnp.int32, 16)

    def chunk_body(k, carry):
        base = k * CH

        def zb(r, carry2):
            for q in range(8):
                rows[r, pl.ds(q * 16, 16)] = zeros16
            return carry2

        lax.fori_loop(0, MB, zb, 0)
        pltpu.sync_copy(rows, chunk_acc.at[pl.ds(s * 256, 128)])
        pltpu.sync_copy(rows, chunk_acc.at[pl.ds(s * 256 + 128, 128)])

        @pl.when(s == 0)
        def _():
            pltpu.sync_copy(rows.at[pl.ds(0, 8)], chunk_acc.at[pl.ds(CH, 8)])

        plsc.subcore_barrier()

        def scan_body(i, off):
            pv = par_res[pl.ds(i * 16, 16)]
            lv = pv - base
            m = (lv >= 0) & (lv < CH)
            tv = tok_res[pl.ds(i * 16, 16)]
            packed = jnp.where(m, lv * 1024 + tv, DUMPP)
            spacked = lax.sort(packed)
            plsc.store_scatter(mpacked, [off + lane16], spacked)
            cnt = plsc.all_reduce_population_count(m)
            return off + cnt[0]

        n_m = lax.fori_loop(0, SCAN_IT, scan_body, jnp.int32(0))

        def padb(j, carry2):
            idx = n_m + j * 16 + lane16
            plsc.store_scatter(mpacked, [idx], dumpv)
            return carry2

        lax.fori_loop(0, MB // 16, padb, 0)

        nb = (n_m + (MB - 1)) // MB

        def bat(b, carry2):
            for q in range(MB // 16):
                pk = mpacked[pl.ds(b * MB + q * 16, 16)]
                pidx_stage[pl.ds(q * 16, 16)] = lax.shift_right_logical(pk, 10)
                tok_stage[pl.ds(q * 16, 16)] = jnp.bitwise_and(pk, 1023)
            pltpu.async_copy(tch_hbm.at[tok_stage], rows, sem).wait()
            pltpu.sync_copy(rows, chunk_acc.at[pidx_stage], add=True)
            return carry2

        lax.fori_loop(0, nb, bat, 0)
        plsc.subcore_barrier()

        pltpu.sync_copy(chunk_acc.at[pl.ds(s * 256, 128)],
                        c2_hbm.at[pl.ds(base + s * 256, 128)])
        pltpu.sync_copy(chunk_acc.at[pl.ds(s * 256 + 128, 128)],
                        c2_hbm.at[pl.ds(base + s * 256 + 128, 128)])
        return carry

    lo = c * SC0_CHUNKS
    hi = SC0_CHUNKS + c * (N_CHUNK - SC0_CHUNKS)
    lax.fori_loop(lo, hi, chunk_body, 0)


def _scatter(tok_pad, par_pad, tch):
    f = functools.partial(
        pl.kernel,
        out_type=jax.ShapeDtypeStruct((PAD_N, HID), jnp.float32),
        mesh=_mesh(),
        compiler_params=_SC_PARAMS,
        scratch_types=[
            pltpu.VMEM((TPT,), jnp.int32),
            pltpu.VMEM((TPT,), jnp.int32),
            pltpu.VMEM((TPT + MB,), jnp.int32),
            pltpu.VMEM((MB,), jnp.int32),
            pltpu.VMEM((MB,), jnp.int32),
            pltpu.VMEM((MB, HID), jnp.float32),
            pltpu.VMEM_SHARED((CHUNK_ROWS, HID), jnp.float32),
            pltpu.SemaphoreType.DMA,
        ],
    )(_scatter_body)
    return f(tok_pad, par_pad, tch)


def _conv_body(tok_hbm, blk_hbm, ts_hbm, c2_hbm, bconv_hbm, out_hbm,
               tokb, blkb, ts_rows, c2_rows, bcv, blk_acc, sem):
    c = lax.axis_index("c")
    s = lax.axis_index("s")
    pltpu.sync_copy(bconv_hbm, bcv)

    zeros16 = jnp.zeros((16,), jnp.float32)

    def zb(r, carry):
        for q in range(8):
            ts_rows[r, pl.ds(q * 16, 16)] = zeros16
        return carry

    lax.fori_loop(0, 128, zb, 0)
    row0 = s * 632
    for j in range(4):
        pltpu.sync_copy(ts_rows, blk_acc.at[pl.ds(row0 + j * 128, 128)])
    pltpu.sync_copy(ts_rows.at[pl.ds(0, 120)],
                    blk_acc.at[pl.ds(row0 + 512, 120)])
    plsc.subcore_barrier()

    tbase = (c * 16 + s) * NPT

    def bat(b, carry):
        nstart = tbase + b * 128
        pltpu.sync_copy(tok_hbm.at[pl.ds(nstart, 128)], tokb)
        pltpu.sync_copy(blk_hbm.at[pl.ds(nstart, 128)], blkb)
        pltpu.async_copy(ts_hbm.at[tokb], ts_rows, sem).wait()
        pltpu.sync_copy(c2_hbm.at[pl.ds(nstart, 128)], c2_rows)

        def relu_b(r, carry2):
            for q in range(8):
                sl = pl.ds(q * 16, 16)
                ts_rows[r, sl] = jnp.maximum(
                    ts_rows[r, sl] + c2_rows[r, sl] + bcv[sl], 0.0)
            return carry2

        lax.fori_loop(0, 128, relu_b, 0)
        pltpu.sync_copy(ts_rows, blk_acc.at[blkb], add=True)
        return carry

    lax.fori_loop(0, NBAT, bat, 0)
    plsc.subcore_barrier()

    for j in range(4):
        pltpu.sync_copy(blk_acc.at[pl.ds(row0 + j * 128, 128)],
                        out_hbm.at[c].at[pl.ds(row0 + j * 128, 128)])
    pltpu.sync_copy(blk_acc.at[pl.ds(row0 + 512, 120)],
                    out_hbm.at[c].at[pl.ds(row0 + 512, 120)])


def _conv(tok_pad, blk_pad, ts, c2, b_conv):
    f = functools.partial(
        pl.kernel,
        out_type=jax.ShapeDtypeStruct((2, BLK_ROWS, HID), jnp.float32),
        mesh=_mesh(),
        compiler_params=_SC_PARAMS,
        scratch_types=[
            pltpu.VMEM((128,), jnp.int32),
            pltpu.VMEM((128,), jnp.int32),
            pltpu.VMEM((128, HID), jnp.float32),
            pltpu.VMEM((128, HID), jnp.float32),
            pltpu.VMEM((HID,), jnp.float32),
            pltpu.VMEM_SHARED((BLK_ROWS, HID), jnp.float32),
            pltpu.SemaphoreType.DMA,
        ],
    )(_conv_body)
    return f(tok_pad, blk_pad, ts, c2, b_conv)


def _block_body(acc_ref, wb_ref, bb_ref, wgi_ref, wgo_ref,
                asgi_ref, adgi_ref, asgo_ref, adgo_ref,
                bf_ref, hgi_aug_ref, hgo_aug_ref,
                o_asgi, o_adgi, o_asgo, o_adgo):
    bs = acc_ref[0] + acc_ref[1]
    bf = jnp.maximum(jnp.dot(bs, wb_ref[...], precision=PREC) + bb_ref[...],
                     0.0)
    bf_ref[...] = bf
    hgi = jnp.dot(bf, wgi_ref[...], precision=PREC)
    hgo = jnp.dot(bf, wgo_ref[...], precision=PREC)
    z = jnp.zeros((bf.shape[0], AUGW - HID // 2), jnp.float32)
    hgi_aug_ref[0] = jnp.concatenate([hgi[:, :64], z], axis=1)
    hgi_aug_ref[1] = jnp.concatenate([hgi[:, 64:], z], axis=1)
    hgo_aug_ref[0] = jnp.concatenate([hgo[:, :64], z], axis=1)
    hgo_aug_ref[1] = jnp.concatenate([hgo[:, 64:], z], axis=1)
    for out_ref, h, a_ref in ((o_asgi, hgi, asgi_ref), (o_adgi, hgi, adgi_ref),
                              (o_asgo, hgo, asgo_ref), (o_adgo, hgo, adgo_ref)):
        out_ref[0] = jnp.dot(h, a_ref[:, 0:2], precision=PREC)
        out_ref[1] = jnp.dot(h, a_ref[:, 2:4], precision=PREC)


def _block(acc, w_block, b_block, w_gi, w_go, a_sgi, a_dgi, a_sgo, a_dgo):
    R = 1000
    grid = (N_BLK // R,)
    full128 = pl.BlockSpec((HID, HID), lambda i: (0, 0))
    alph = pl.BlockSpec((HID, HEADS), lambda i: (0, 0))
    alph_out = pl.BlockSpec((2, R, 2), lambda i: (0, i, 0))
    aug_out = pl.BlockSpec((2, R, AUGW), lambda i: (0, i, 0))
    return pl.pallas_call(
        _block_body,
        grid=grid,
        in_specs=[
            pl.BlockSpec((2, R, HID), lambda i: (0, i, 0)),
            full128,
            pl.BlockSpec((1, HID), lambda i: (0, 0)),
            full128, full128,
            alph, alph, alph, alph,
        ],
        out_specs=[
            pl.BlockSpec((R, HID), lambda i: (i, 0)),
            aug_out, aug_out,
            alph_out, alph_out, alph_out, alph_out,
        ],
        out_shape=(
            jax.ShapeDtypeStruct((N_BLK, HID), jnp.float32),
            jax.ShapeDtypeStruct((2, N_BLK, AUGW), jnp.float32),
            jax.ShapeDtypeStruct((2, N_BLK, AUGW), jnp.float32),
            jax.ShapeDtypeStruct((2, N_BLK, 2), jnp.float32),
            jax.ShapeDtypeStruct((2, N_BLK, 2), jnp.float32),
            jax.ShapeDtypeStruct((2, N_BLK, 2), jnp.float32),
            jax.ShapeDtypeStruct((2, N_BLK, 2), jnp.float32),
        ),
    )(acc, w_block, b_block, w_gi, w_go, a_sgi, a_dgi, a_sgo, a_dgo)


def _gat_body(src_hbm, dst_hbm, et_hbm, asrc_hbm, adst_hbm, etb_hbm, haug_hbm,
              out_hbm, asrc_res, adst_res, etb_res, sb, db, tb, sidx,
              wb0, wb1, rows, out_acc, sem):
    c = lax.axis_index("c")
    s = lax.axis_index("s")
    pltpu.sync_copy(asrc_hbm.at[c], asrc_res)
    pltpu.sync_copy(adst_hbm.at[c], adst_res)
    pltpu.sync_copy(etb_hbm.at[c], etb_res)
    himask = jnp.full((16,), -65536, jnp.int32)

    zeros16 = jnp.zeros((16,), jnp.float32)

    def zb(r, carry):
        for q in range(AUGW // 16):
            rows[r, pl.ds(q * 16, 16)] = zeros16
        return carry

    lax.fori_loop(0, EB, zb, 0)
    row0 = s * 632
    for j in range(8):
        pltpu.sync_copy(rows.at[pl.ds(0, 79)],
                        out_acc.at[pl.ds(row0 + j * 79, 79)])
    plsc.subcore_barrier()

    coff = c * N_BLK
    lane = lax.iota(jnp.int32, 16)

    def bat(b, carry):
        ebase = s * EPT + b * EB
        pltpu.sync_copy(src_hbm.at[pl.ds(ebase, EB)], sb)
        pltpu.sync_copy(dst_hbm.at[pl.ds(ebase, EB)], db)
        pltpu.sync_copy(et_hbm.at[pl.ds(ebase, EB)], tb)
        for k in range(EB // 16):
            sl = pl.ds(k * 16, 16)
            sv = sb[sl]
            dv = db[sl]
            tv = tb[sl]
            p1 = plsc.load_gather(asrc_res, [sv])
            p2 = plsc.load_gather(adst_res, [dv])
            p3 = plsc.load_gather(etb_res, [tv])
            for h in range(2):
                if h == 0:
                    a1 = plsc.bitcast(jnp.bitwise_and(p1, himask), jnp.float32)
                    a2 = plsc.bitcast(jnp.bitwise_and(p2, himask), jnp.float32)
                    a3 = plsc.bitcast(jnp.bitwise_and(p3, himask), jnp.float32)
                else:
                    a1 = plsc.bitcast(lax.shift_left(p1, 16), jnp.float32)
                    a2 = plsc.bitcast(lax.shift_left(p2, 16), jnp.float32)
                    a3 = plsc.bitcast(lax.shift_left(p3, 16), jnp.float32)
                e = a1 + a2 + a3
                e = jnp.where(e >= 0.0, e, 0.2 * e)
                w = jnp.exp(e)
                if h == 0:
                    wb0[sl] = w
                else:
                    wb1[sl] = w
            sidx[sl] = sv + coff
        pltpu.async_copy(haug_hbm.at[sidx], rows, sem).wait()

        def scale(k, carry2):
            wv0 = wb0[pl.ds(k * 16, 16)]
            wv1 = wb1[pl.ds(k * 16, 16)]
            for j16 in range(16):
                j = k * 16 + j16
                w0 = wv0[j16]
                w1 = wv1[j16]
                for q in range(2):
                    sl = pl.ds(q * 16, 16)
                    rows[j, sl] = rows[j, sl] * w0
                for q in range(2, 4):
                    sl = pl.ds(q * 16, 16)
                    rows[j, sl] = rows[j, sl] * w1
                rows[j, pl.ds(64, 16)] = jnp.where(
                    lane == 0, w0, jnp.where(lane == 1, w1, 0.0))
            return carry2

        lax.fori_loop(0, EB // 16, scale, 0)
        pltpu.sync_copy(rows, out_acc.at[db], add=True)
        return carry

    lax.fori_loop(0, NEB, bat, 0)
    plsc.subcore_barrier()

    for j in range(4):
        pltpu.sync_copy(out_acc.at[pl.ds(row0 + j * 128, 128)],
                        out_hbm.at[c].at[pl.ds(row0 + j * 128, 128)])
    pltpu.sync_copy(out_acc.at[pl.ds(row0 + 512, 120)],
                    out_hbm.at[c].at[pl.ds(row0 + 512, 120)])


def _gat(srcx, dstx, etx, asrc_sc, adst_sc, etb_sc, haug):
    f = functools.partial(
        pl.kernel,
        out_type=jax.ShapeDtypeStruct((2, BLK_ROWS, AUGW), jnp.float32),
        mesh=_mesh(),
        compiler_params=_SC_PARAMS,
        scratch_types=[
            pltpu.VMEM((N_BLK,), jnp.int32),
            pltpu.VMEM((N_BLK,), jnp.int32),
            pltpu.VMEM((8,), jnp.int32),
            pltpu.VMEM((EB,), jnp.int32),
            pltpu.VMEM((EB,), jnp.int32),
            pltpu.VMEM((EB,), jnp.int32),
            pltpu.VMEM((EB,), jnp.int32),
            pltpu.VMEM((EB,), jnp.float32),
            pltpu.VMEM((EB,), jnp.float32),
            pltpu.VMEM((EB, AUGW), jnp.float32),
            pltpu.VMEM_SHARED((BLK_ROWS, AUGW), jnp.float32),
            pltpu.SemaphoreType.DMA,
        ],
    )(_gat_body)
    return f(srcx, dstx, etx, asrc_sc, adst_sc, etb_sc, haug)


def _final_body(bf_ref, ain_ref, aout_ref, wm_ref, bm_ref, lab_ref,
                sig_ref, loss_ref, pooled_ref, macc):
    i = pl.program_id(0)

    @pl.when(i == 0)
    def _():
        macc[...] = jnp.full((1, HID), -jnp.inf, jnp.float32)

    parts = []
    for c in range(2):
        for g in range(2):
            num_i = ain_ref[c, :, g * DH:(g + 1) * DH]
            den_i = ain_ref[c, :, 64 + g:65 + g]
            num_o = aout_ref[c, :, g * DH:(g + 1) * DH]
            den_o = aout_ref[c, :, 64 + g:65 + g]
            parts.append(num_i / (den_i + 1e-9) + num_o / (den_o + 1e-9))
    fea = bf_ref[...] + jnp.concatenate(parts, axis=1)
    macc[...] = jnp.maximum(macc[...], jnp.max(fea, axis=0, keepdims=True))

    @pl.when(i == pl.num_programs(0) - 1)
    def _():
        pooled = macc[...]
        logits = jnp.dot(pooled, wm_ref[...], precision=PREC) + bm_ref[...]
        pooled_ref[...] = pooled
        sig_ref[...] = 1.0 / (1.0 + jnp.exp(-logits))
        l = logits[0, 0]
        y = lab_ref[0, 0]
        loss_ref[...] = (jnp.maximum(l, 0.0) - l * y +
                         jnp.log1p(jnp.exp(-jnp.abs(l)))).reshape(1, 1)


def _final(bf, accin, accout, w_mlp, b_mlp, lab):
    R = 1000
    grid = (N_BLK // R,)
    return pl.pallas_call(
        _final_body,
        grid=grid,
        in_specs=[
            pl.BlockSpec((R, HID), lambda i: (i, 0)),
            pl.BlockSpec((2, R, AUGW), lambda i: (0, i, 0)),
            pl.BlockSpec((2, R, AUGW), lambda i: (0, i, 0)),
            pl.BlockSpec((HID, 1), lambda i: (0, 0)),
            pl.BlockSpec((1, 1), lambda i: (0, 0)),
            pl.BlockSpec((1, 1), lambda i: (0, 0)),
        ],
        out_specs=[
            pl.BlockSpec((1, 1), lambda i: (0, 0)),
            pl.BlockSpec((1, 1), lambda i: (0, 0)),
            pl.BlockSpec((1, HID), lambda i: (0, 0)),
        ],
        out_shape=(
            jax.ShapeDtypeStruct((1, 1), jnp.float32),
            jax.ShapeDtypeStruct((1, 1), jnp.float32),
            jax.ShapeDtypeStruct((1, HID), jnp.float32),
        ),
        scratch_shapes=[pltpu.VMEM((1, HID), jnp.float32)],
    )(bf, accin, accout, w_mlp, b_mlp, lab)


def _pack_pair(a):
    bb = lax.bitcast_convert_type(a.astype(jnp.bfloat16),
                                  jnp.uint16).astype(jnp.uint32)
    return (jnp.left_shift(bb[..., 0], 16) | bb[..., 1]).astype(jnp.int32)


def _alpha_mat(a):
    m = jnp.zeros((HID, HEADS), jnp.float32)
    for h in range(HEADS):
        m = m.at[h * DH:(h + 1) * DH, h].set(a[h])
    return m


def kernel(ast_tokens, ast_parent, emb_block_id, cfg_edge_index, cfg_edge_type,
           label, embed_table, W_self, W_child, b_conv, W_block, b_block,
           W_gi, a_src_gi, a_dst_gi, et_gi,
           W_go, a_src_go, a_dst_go, et_go,
           W_mlp, b_mlp):
    i32 = jnp.int32
    npad = PAD_N - N_AST
    tok_pad = jnp.concatenate([ast_tokens.astype(i32),
                               jnp.zeros((npad,), i32)])
    par_pad = jnp.concatenate([ast_parent.astype(i32),
                               jnp.full((npad,), -1, i32)])
    blk_pad = jnp.concatenate([emb_block_id.astype(i32),
                               jnp.full((npad,), N_BLK, i32)])

    ts, tch = _prep(embed_table, W_self, W_child)
    c2 = _scatter(tok_pad, par_pad, tch)
    acc = _conv(tok_pad, blk_pad, ts, c2, b_conv)

    bf, haug_gi, haug_go, asgi, adgi, asgo, adgo = _block(
        acc, W_block, b_block.reshape(1, HID), W_gi, W_go,
        _alpha_mat(a_src_gi), _alpha_mat(a_dst_gi),
        _alpha_mat(a_src_go), _alpha_mat(a_dst_go))

    srcx = cfg_edge_index[0].astype(i32)
    dstx = cfg_edge_index[1].astype(i32)
    etx = cfg_edge_type.astype(i32)
    asgi = _pack_pair(asgi)
    adgi = _pack_pair(adgi)
    asgo = _pack_pair(asgo)
    adgo = _pack_pair(adgo)
    etb_gi = _pack_pair(jnp.pad(
        et_gi.reshape(3, 2, 2).transpose(1, 0, 2), ((0, 0), (0, 5), (0, 0))))
    etb_go = _pack_pair(jnp.pad(
        et_go.reshape(3, 2, 2).transpose(1, 0, 2), ((0, 0), (0, 5), (0, 0))))

    accin = _gat(srcx, dstx, etx, asgi, adgi, etb_gi,
                 haug_gi.reshape(2 * N_BLK, AUGW))
    accout = _gat(dstx, srcx, etx, asgo, adgo, etb_go,
                  haug_go.reshape(2 * N_BLK, AUGW))

    sig, loss, pooled = _final(bf, accin, accout, W_mlp,
                               b_mlp.reshape(1, 1), label.reshape(1, 1))
    return (sig, loss.reshape(()), pooled)


# speedup vs baseline: 27.1979x; 1.1751x over previous
import functools

import jax
import jax.numpy as jnp
from jax import lax
from jax.experimental import pallas as pl
from jax.experimental.pallas import tpu as pltpu
from jax.experimental.pallas import tpu_sc as plsc

N_AST = 100000
N_BLK = 10000
E_CFG = 320000
TOKEN_SIZE = 1000
FEAT = 128
HID = 128
HEADS = 4
DH = HID // HEADS

PAD_N = 102400
N_CHUNK = 25
CH = 4096
SC0_CHUNKS = 13
CHUNK_ROWS = CH + 8
BLK_ROWS = 10112
TPT = PAD_N // 16
SCAN_IT = TPT // 16
MB = 128
NPT = PAD_N // 32
NBAT = NPT // 128
EPT = E_CFG // 16
EB = 80
NEB = EPT // EB
AUGW = 128
DUMPP = CH * 1024
PREC = lax.Precision.HIGHEST


def _mesh():
    return plsc.VectorSubcoreMesh(core_axis_name="c", subcore_axis_name="s",
                                  num_cores=2, num_subcores=16)


_SC_PARAMS = pltpu.CompilerParams(needs_layout_passes=False)


def _prep_body(emb_ref, ws_ref, wc_ref, ts_ref, tch_ref):
    ts_ref[...] = jnp.dot(emb_ref[...], ws_ref[...], precision=PREC)
    tch_ref[...] = jnp.dot(emb_ref[...], wc_ref[...], precision=PREC)


def _prep(embed_table, w_self, w_child):
    return pl.pallas_call(
        _prep_body,
        out_shape=(
            jax.ShapeDtypeStruct((TOKEN_SIZE, HID), jnp.float32),
            jax.ShapeDtypeStruct((TOKEN_SIZE, HID), jnp.float32),
        ),
    )(embed_table, w_self, w_child)


def _scatter_body(tok_hbm, par_hbm, tch_hbm, c2_hbm,
                  par_res, tok_res, mpacked, pidx_stage, tok_stage, rows,
                  chunk_acc, sem):
    c = lax.axis_index("c")
    s = lax.axis_index("s")
    pltpu.sync_copy(par_hbm.at[pl.ds(s * TPT, TPT)], par_res)
    pltpu.sync_copy(tok_hbm.at[pl.ds(s * TPT, TPT)], tok_res)

    zeros16 = jnp.zeros((16,), jnp.float32)
    dumpv = jnp.full((16,), DUMPP, jnp.int32)
    lane16 = lax.iota(jnp.int32, 16)

    def chunk_body(k, carry):
        base = k * CH

        def zb(r, carry2):
            for q in range(8):
                rows[r, pl.ds(q * 16, 16)] = zeros16
            return carry2

        lax.fori_loop(0, MB, zb, 0)
        pltpu.sync_copy(rows, chunk_acc.at[pl.ds(s * 256, 128)])
        pltpu.sync_copy(rows, chunk_acc.at[pl.ds(s * 256 + 128, 128)])

        @pl.when(s == 0)
        def _():
            pltpu.sync_copy(rows.at[pl.ds(0, 8)], chunk_acc.at[pl.ds(CH, 8)])

        plsc.subcore_barrier()

        def scan_body(i, off):
            pv = par_res[pl.ds(i * 16, 16)]
            lv = pv - base
            m = (lv >= 0) & (lv < CH)
            tv = tok_res[pl.ds(i * 16, 16)]
            packed = jnp.where(m, lv * 1024 + tv, DUMPP)
            spacked = lax.sort(packed)
            plsc.store_scatter(mpacked, [off + lane16], spacked)
            cnt = plsc.all_reduce_population_count(m)
            return off + cnt[0]

        n_m = lax.fori_loop(0, SCAN_IT, scan_body, jnp.int32(0))

        def padb(j, carry2):
            idx = n_m + j * 16 + lane16
            plsc.store_scatter(mpacked, [idx], dumpv)
            return carry2

        lax.fori_loop(0, MB // 16, padb, 0)

        nb = (n_m + (MB - 1)) // MB

        def bat(b, carry2):
            for q in range(MB // 16):
                pk = mpacked[pl.ds(b * MB + q * 16, 16)]
                pidx_stage[pl.ds(q * 16, 16)] = lax.shift_right_logical(pk, 10)
                tok_stage[pl.ds(q * 16, 16)] = jnp.bitwise_and(pk, 1023)
            pltpu.async_copy(tch_hbm.at[tok_stage], rows, sem).wait()
            pltpu.sync_copy(rows, chunk_acc.at[pidx_stage], add=True)
            return carry2

        lax.fori_loop(0, nb, bat, 0)
        plsc.subcore_barrier()

        pltpu.sync_copy(chunk_acc.at[pl.ds(s * 256, 128)],
                        c2_hbm.at[pl.ds(base + s * 256, 128)])
        pltpu.sync_copy(chunk_acc.at[pl.ds(s * 256 + 128, 128)],
                        c2_hbm.at[pl.ds(base + s * 256 + 128, 128)])
        return carry

    lo = c * SC0_CHUNKS
    hi = SC0_CHUNKS + c * (N_CHUNK - SC0_CHUNKS)
    lax.fori_loop(lo, hi, chunk_body, 0)


def _scatter(tok_pad, par_pad, tch):
    f = functools.partial(
        pl.kernel,
        out_type=jax.ShapeDtypeStruct((PAD_N, HID), jnp.float32),
        mesh=_mesh(),
        compiler_params=_SC_PARAMS,
        scratch_types=[
            pltpu.VMEM((TPT,), jnp.int32),
            pltpu.VMEM((TPT,), jnp.int32),
            pltpu.VMEM((TPT + MB,), jnp.int32),
            pltpu.VMEM((MB,), jnp.int32),
            pltpu.VMEM((MB,), jnp.int32),
            pltpu.VMEM((MB, HID), jnp.float32),
            pltpu.VMEM_SHARED((CHUNK_ROWS, HID), jnp.float32),
            pltpu.SemaphoreType.DMA,
        ],
    )(_scatter_body)
    return f(tok_pad, par_pad, tch)


def _conv_body(tok_hbm, blk_hbm, ts_hbm, c2_hbm, bconv_hbm, out_hbm,
               tokb, blkb, ts_rows, c2_rows, bcv, blk_acc, sem):
    c = lax.axis_index("c")
    s = lax.axis_index("s")
    pltpu.sync_copy(bconv_hbm, bcv)

    zeros16 = jnp.zeros((16,), jnp.float32)

    def zb(r, carry):
        for q in range(8):
            ts_rows[r, pl.ds(q * 16, 16)] = zeros16
        return carry

    lax.fori_loop(0, 128, zb, 0)
    row0 = s * 632
    for j in range(4):
        pltpu.sync_copy(ts_rows, blk_acc.at[pl.ds(row0 + j * 128, 128)])
    pltpu.sync_copy(ts_rows.at[pl.ds(0, 120)],
                    blk_acc.at[pl.ds(row0 + 512, 120)])
    plsc.subcore_barrier()

    tbase = (c * 16 + s) * NPT

    def bat(b, carry):
        nstart = tbase + b * 128
        pltpu.sync_copy(tok_hbm.at[pl.ds(nstart, 128)], tokb)
        pltpu.sync_copy(blk_hbm.at[pl.ds(nstart, 128)], blkb)
        pltpu.async_copy(ts_hbm.at[tokb], ts_rows, sem).wait()
        pltpu.sync_copy(c2_hbm.at[pl.ds(nstart, 128)], c2_rows)

        def relu_b(r, carry2):
            for q in range(8):
                sl = pl.ds(q * 16, 16)
                ts_rows[r, sl] = jnp.maximum(
                    ts_rows[r, sl] + c2_rows[r, sl] + bcv[sl], 0.0)
            return carry2

        lax.fori_loop(0, 128, relu_b, 0)
        pltpu.sync_copy(ts_rows, blk_acc.at[blkb], add=True)
        return carry

    lax.fori_loop(0, NBAT, bat, 0)
    plsc.subcore_barrier()

    for j in range(4):
        pltpu.sync_copy(blk_acc.at[pl.ds(row0 + j * 128, 128)],
                        out_hbm.at[c].at[pl.ds(row0 + j * 128, 128)])
    pltpu.sync_copy(blk_acc.at[pl.ds(row0 + 512, 120)],
                    out_hbm.at[c].at[pl.ds(row0 + 512, 120)])


def _conv(tok_pad, blk_pad, ts, c2, b_conv):
    f = functools.partial(
        pl.kernel,
        out_type=jax.ShapeDtypeStruct((2, BLK_ROWS, HID), jnp.float32),
        mesh=_mesh(),
        compiler_params=_SC_PARAMS,
        scratch_types=[
            pltpu.VMEM((128,), jnp.int32),
            pltpu.VMEM((128,), jnp.int32),
            pltpu.VMEM((128, HID), jnp.float32),
            pltpu.VMEM((128, HID), jnp.float32),
            pltpu.VMEM((HID,), jnp.float32),
            pltpu.VMEM_SHARED((BLK_ROWS, HID), jnp.float32),
            pltpu.SemaphoreType.DMA,
        ],
    )(_conv_body)
    return f(tok_pad, blk_pad, ts, c2, b_conv)


def _block_body(acc_ref, wb_ref, bb_ref, wgi_ref, wgo_ref,
                asgi_ref, adgi_ref, asgo_ref, adgo_ref,
                bf_ref, hgi_aug_ref, hgo_aug_ref,
                o_asgi, o_adgi, o_asgo, o_adgo):
    bs = acc_ref[0] + acc_ref[1]
    bf = jnp.maximum(jnp.dot(bs, wb_ref[...], precision=PREC) + bb_ref[...],
                     0.0)
    bf_ref[...] = bf
    hgi = jnp.dot(bf, wgi_ref[...], precision=PREC)
    hgo = jnp.dot(bf, wgo_ref[...], precision=PREC)
    z = jnp.zeros((bf.shape[0], AUGW - HID // 2), jnp.float32)
    hgi_aug_ref[0] = jnp.concatenate([hgi[:, :64], z], axis=1)
    hgi_aug_ref[1] = jnp.concatenate([hgi[:, 64:], z], axis=1)
    hgo_aug_ref[0] = jnp.concatenate([hgo[:, :64], z], axis=1)
    hgo_aug_ref[1] = jnp.concatenate([hgo[:, 64:], z], axis=1)
    for out_ref, h, a_ref in ((o_asgi, hgi, asgi_ref), (o_adgi, hgi, adgi_ref),
                              (o_asgo, hgo, asgo_ref), (o_adgo, hgo, adgo_ref)):
        out_ref[0] = jnp.dot(h, a_ref[:, 0:2], precision=PREC)
        out_ref[1] = jnp.dot(h, a_ref[:, 2:4], precision=PREC)


def _block(acc, w_block, b_block, w_gi, w_go, a_sgi, a_dgi, a_sgo, a_dgo):
    R = 1000
    grid = (N_BLK // R,)
    full128 = pl.BlockSpec((HID, HID), lambda i: (0, 0))
    alph = pl.BlockSpec((HID, HEADS), lambda i: (0, 0))
    alph_out = pl.BlockSpec((2, R, 2), lambda i: (0, i, 0))
    aug_out = pl.BlockSpec((2, R, AUGW), lambda i: (0, i, 0))
    return pl.pallas_call(
        _block_body,
        grid=grid,
        in_specs=[
            pl.BlockSpec((2, R, HID), lambda i: (0, i, 0)),
            full128,
            pl.BlockSpec((1, HID), lambda i: (0, 0)),
            full128, full128,
            alph, alph, alph, alph,
        ],
        out_specs=[
            pl.BlockSpec((R, HID), lambda i: (i, 0)),
            aug_out, aug_out,
            alph_out, alph_out, alph_out, alph_out,
        ],
        out_shape=(
            jax.ShapeDtypeStruct((N_BLK, HID), jnp.float32),
            jax.ShapeDtypeStruct((2, N_BLK, AUGW), jnp.float32),
            jax.ShapeDtypeStruct((2, N_BLK, AUGW), jnp.float32),
            jax.ShapeDtypeStruct((2, N_BLK, 2), jnp.float32),
            jax.ShapeDtypeStruct((2, N_BLK, 2), jnp.float32),
            jax.ShapeDtypeStruct((2, N_BLK, 2), jnp.float32),
            jax.ShapeDtypeStruct((2, N_BLK, 2), jnp.float32),
        ),
    )(acc, w_block, b_block, w_gi, w_go, a_sgi, a_dgi, a_sgo, a_dgo)


def _gat_body(src_hbm, dst_hbm, et_hbm, asrc_hbm, adst_hbm, etb_hbm, haug_hbm,
              out_hbm, asrc_res, adst_res, etb_res, sb2, db2, tb2, sidx2,
              wb02, wb12, rows2, out_acc, sem0, sem1):
    c = lax.axis_index("c")
    s = lax.axis_index("s")
    pltpu.sync_copy(asrc_hbm.at[c], asrc_res)
    pltpu.sync_copy(adst_hbm.at[c], adst_res)
    pltpu.sync_copy(etb_hbm.at[c], etb_res)
    himask = jnp.full((16,), -65536, jnp.int32)

    zeros16 = jnp.zeros((16,), jnp.float32)

    def zb(r, carry):
        for q in range(AUGW // 16):
            rows2[0, r, pl.ds(q * 16, 16)] = zeros16
        return carry

    lax.fori_loop(0, EB, zb, 0)
    row0 = s * 632
    for j in range(8):
        pltpu.sync_copy(rows2.at[0].at[pl.ds(0, 79)],
                        out_acc.at[pl.ds(row0 + j * 79, 79)])
    plsc.subcore_barrier()

    coff = c * N_BLK
    lane = lax.iota(jnp.int32, 16)
    sems = (sem0, sem1)

    def stage(b, i):
        ebase = s * EPT + b * EB
        pltpu.sync_copy(src_hbm.at[pl.ds(ebase, EB)], sb2.at[i])
        pltpu.sync_copy(dst_hbm.at[pl.ds(ebase, EB)], db2.at[i])
        pltpu.sync_copy(et_hbm.at[pl.ds(ebase, EB)], tb2.at[i])
        for k in range(EB // 16):
            sl = pl.ds(k * 16, 16)
            sv = sb2[i, sl]
            dv = db2[i, sl]
            tv = tb2[i, sl]
            p1 = plsc.load_gather(asrc_res, [sv])
            p2 = plsc.load_gather(adst_res, [dv])
            p3 = plsc.load_gather(etb_res, [tv])
            for h in range(2):
                if h == 0:
                    a1 = plsc.bitcast(jnp.bitwise_and(p1, himask), jnp.float32)
                    a2 = plsc.bitcast(jnp.bitwise_and(p2, himask), jnp.float32)
                    a3 = plsc.bitcast(jnp.bitwise_and(p3, himask), jnp.float32)
                else:
                    a1 = plsc.bitcast(lax.shift_left(p1, 16), jnp.float32)
                    a2 = plsc.bitcast(lax.shift_left(p2, 16), jnp.float32)
                    a3 = plsc.bitcast(lax.shift_left(p3, 16), jnp.float32)
                e = a1 + a2 + a3
                e = jnp.where(e >= 0.0, e, 0.2 * e)
                w = jnp.exp(e)
                if h == 0:
                    wb02[i, sl] = w
                else:
                    wb12[i, sl] = w
            sidx2[i, sl] = sv + coff
        pltpu.async_copy(haug_hbm.at[sidx2.at[i]], rows2.at[i], sems[i])

    def drain(i):
        pltpu.make_async_copy(haug_hbm.at[sidx2.at[i]], rows2.at[i],
                              sems[i]).wait()

        def scale(k, carry2):
            wv0 = wb02[i, pl.ds(k * 16, 16)]
            wv1 = wb12[i, pl.ds(k * 16, 16)]
            for j16 in range(16):
                j = k * 16 + j16
                w0 = wv0[j16]
                w1 = wv1[j16]
                for q in range(2):
                    sl = pl.ds(q * 16, 16)
                    rows2[i, j, sl] = rows2[i, j, sl] * w0
                for q in range(2, 4):
                    sl = pl.ds(q * 16, 16)
                    rows2[i, j, sl] = rows2[i, j, sl] * w1
                rows2[i, j, pl.ds(64, 16)] = jnp.where(
                    lane == 0, w0, jnp.where(lane == 1, w1, 0.0))
            return carry2

        lax.fori_loop(0, EB // 16, scale, 0)
        pltpu.sync_copy(rows2.at[i], out_acc.at[db2.at[i]], add=True)

    stage(0, 0)
    NPAIR = NEB // 2

    def bat2(p, carry):
        stage(2 * p + 1, 1)
        drain(0)

        @pl.when(p < NPAIR - 1)
        def _():
            stage(2 * p + 2, 0)

        drain(1)
        return carry

    lax.fori_loop(0, NPAIR, bat2, 0)
    plsc.subcore_barrier()

    for j in range(4):
        pltpu.sync_copy(out_acc.at[pl.ds(row0 + j * 128, 128)],
                        out_hbm.at[c].at[pl.ds(row0 + j * 128, 128)])
    pltpu.sync_copy(out_acc.at[pl.ds(row0 + 512, 120)],
                    out_hbm.at[c].at[pl.ds(row0 + 512, 120)])


def _gat(srcx, dstx, etx, asrc_sc, adst_sc, etb_sc, haug):
    f = functools.partial(
        pl.kernel,
        out_type=jax.ShapeDtypeStruct((2, BLK_ROWS, AUGW), jnp.float32),
        mesh=_mesh(),
        compiler_params=_SC_PARAMS,
        scratch_types=[
            pltpu.VMEM((N_BLK,), jnp.int32),
            pltpu.VMEM((N_BLK,), jnp.int32),
            pltpu.VMEM((8,), jnp.int32),
            pltpu.VMEM((2, EB), jnp.int32),
            pltpu.VMEM((2, EB), jnp.int32),
            pltpu.VMEM((2, EB), jnp.int32),
            pltpu.VMEM((2, EB), jnp.int32),
            pltpu.VMEM((2, EB), jnp.float32),
            pltpu.VMEM((2, EB), jnp.float32),
            pltpu.VMEM((2, EB, AUGW), jnp.float32),
            pltpu.VMEM_SHARED((BLK_ROWS, AUGW), jnp.float32),
            pltpu.SemaphoreType.DMA,
            pltpu.SemaphoreType.DMA,
        ],
    )(_gat_body)
    return f(srcx, dstx, etx, asrc_sc, adst_sc, etb_sc, haug)


def _final_body(bf_ref, ain_ref, aout_ref, wm_ref, bm_ref, lab_ref,
                sig_ref, loss_ref, pooled_ref, macc):
    i = pl.program_id(0)

    @pl.when(i == 0)
    def _():
        macc[...] = jnp.full((1, HID), -jnp.inf, jnp.float32)

    parts = []
    for c in range(2):
        for g in range(2):
            num_i = ain_ref[c, :, g * DH:(g + 1) * DH]
            den_i = ain_ref[c, :, 64 + g:65 + g]
            num_o = aout_ref[c, :, g * DH:(g + 1) * DH]
            den_o = aout_ref[c, :, 64 + g:65 + g]
            parts.append(num_i / (den_i + 1e-9) + num_o / (den_o + 1e-9))
    fea = bf_ref[...] + jnp.concatenate(parts, axis=1)
    macc[...] = jnp.maximum(macc[...], jnp.max(fea, axis=0, keepdims=True))

    @pl.when(i == pl.num_programs(0) - 1)
    def _():
        pooled = macc[...]
        logits = jnp.dot(pooled, wm_ref[...], precision=PREC) + bm_ref[...]
        pooled_ref[...] = pooled
        sig_ref[...] = 1.0 / (1.0 + jnp.exp(-logits))
        l = logits[0, 0]
        y = lab_ref[0, 0]
        loss_ref[...] = (jnp.maximum(l, 0.0) - l * y +
                         jnp.log1p(jnp.exp(-jnp.abs(l)))).reshape(1, 1)


def _final(bf, accin, accout, w_mlp, b_mlp, lab):
    R = 1000
    grid = (N_BLK // R,)
    return pl.pallas_call(
        _final_body,
        grid=grid,
        in_specs=[
            pl.BlockSpec((R, HID), lambda i: (i, 0)),
            pl.BlockSpec((2, R, AUGW), lambda i: (0, i, 0)),
            pl.BlockSpec((2, R, AUGW), lambda i: (0, i, 0)),
            pl.BlockSpec((HID, 1), lambda i: (0, 0)),
            pl.BlockSpec((1, 1), lambda i: (0, 0)),
            pl.BlockSpec((1, 1), lambda i: (0, 0)),
        ],
        out_specs=[
            pl.BlockSpec((1, 1), lambda i: (0, 0)),
            pl.BlockSpec((1, 1), lambda i: (0, 0)),
            pl.BlockSpec((1, HID), lambda i: (0, 0)),
        ],
        out_shape=(
            jax.ShapeDtypeStruct((1, 1), jnp.float32),
            jax.ShapeDtypeStruct((1, 1), jnp.float32),
            jax.ShapeDtypeStruct((1, HID), jnp.float32),
        ),
        scratch_shapes=[pltpu.VMEM((1, HID), jnp.float32)],
    )(bf, accin, accout, w_mlp, b_mlp, lab)


def _pack_pair(a):
    bb = lax.bitcast_convert_type(a.astype(jnp.bfloat16),
                                  jnp.uint16).astype(jnp.uint32)
    return (jnp.left_shift(bb[..., 0], 16) | bb[..., 1]).astype(jnp.int32)


def _alpha_mat(a):
    m = jnp.zeros((HID, HEADS), jnp.float32)
    for h in range(HEADS):
        m = m.at[h * DH:(h + 1) * DH, h].set(a[h])
    return m


def kernel(ast_tokens, ast_parent, emb_block_id, cfg_edge_index, cfg_edge_type,
           label, embed_table, W_self, W_child, b_conv, W_block, b_block,
           W_gi, a_src_gi, a_dst_gi, et_gi,
           W_go, a_src_go, a_dst_go, et_go,
           W_mlp, b_mlp):
    i32 = jnp.int32
    npad = PAD_N - N_AST
    tok_pad = jnp.concatenate([ast_tokens.astype(i32),
                               jnp.zeros((npad,), i32)])
    par_pad = jnp.concatenate([ast_parent.astype(i32),
                               jnp.full((npad,), -1, i32)])
    blk_pad = jnp.concatenate([emb_block_id.astype(i32),
                               jnp.full((npad,), N_BLK, i32)])

    ts, tch = _prep(embed_table, W_self, W_child)
    c2 = _scatter(tok_pad, par_pad, tch)
    acc = _conv(tok_pad, blk_pad, ts, c2, b_conv)

    bf, haug_gi, haug_go, asgi, adgi, asgo, adgo = _block(
        acc, W_block, b_block.reshape(1, HID), W_gi, W_go,
        _alpha_mat(a_src_gi), _alpha_mat(a_dst_gi),
        _alpha_mat(a_src_go), _alpha_mat(a_dst_go))

    srcx = cfg_edge_index[0].astype(i32)
    dstx = cfg_edge_index[1].astype(i32)
    etx = cfg_edge_type.astype(i32)
    asgi = _pack_pair(asgi)
    adgi = _pack_pair(adgi)
    asgo = _pack_pair(asgo)
    adgo = _pack_pair(adgo)
    etb_gi = _pack_pair(jnp.pad(
        et_gi.reshape(3, 2, 2).transpose(1, 0, 2), ((0, 0), (0, 5), (0, 0))))
    etb_go = _pack_pair(jnp.pad(
        et_go.reshape(3, 2, 2).transpose(1, 0, 2), ((0, 0), (0, 5), (0, 0))))

    accin = _gat(srcx, dstx, etx, asgi, adgi, etb_gi,
                 haug_gi.reshape(2 * N_BLK, AUGW))
    accout = _gat(dstx, srcx, etx, asgo, adgo, etb_go,
                  haug_go.reshape(2 * N_BLK, AUGW))

    sig, loss, pooled = _final(bf, accin, accout, W_mlp,
                               b_mlp.reshape(1, 1), label.reshape(1, 1))
    return (sig, loss.reshape(()), pooled)


# speedup vs baseline: 27.3815x; 1.0068x over previous
import functools

import jax
import jax.numpy as jnp
from jax import lax
from jax.experimental import pallas as pl
from jax.experimental.pallas import tpu as pltpu
from jax.experimental.pallas import tpu_sc as plsc

N_AST = 100000
N_BLK = 10000
E_CFG = 320000
TOKEN_SIZE = 1000
FEAT = 128
HID = 128
HEADS = 4
DH = HID // HEADS

PAD_N = 102400
N_CHUNK = 25
CH = 4096
SC0_CHUNKS = 13
CHUNK_ROWS = CH + 8
BLK_ROWS = 10112
TPT = PAD_N // 16
SCAN_IT = TPT // 16
MB = 128
NPT = PAD_N // 32
NBAT = NPT // 128
EPT = E_CFG // 16
EB = 80
NEB = EPT // EB
AUGW = 128
DUMPP = CH * 1024
PREC = lax.Precision.HIGHEST


def _mesh():
    return plsc.VectorSubcoreMesh(core_axis_name="c", subcore_axis_name="s",
                                  num_cores=2, num_subcores=16)


_SC_PARAMS = pltpu.CompilerParams(needs_layout_passes=False)


def _prep_body(emb_ref, ws_ref, wc_ref, ts_ref, tch_ref):
    ts_ref[...] = jnp.dot(emb_ref[...], ws_ref[...], precision=PREC)
    tch_ref[...] = jnp.dot(emb_ref[...], wc_ref[...], precision=PREC)


def _prep(embed_table, w_self, w_child):
    return pl.pallas_call(
        _prep_body,
        out_shape=(
            jax.ShapeDtypeStruct((TOKEN_SIZE, HID), jnp.float32),
            jax.ShapeDtypeStruct((TOKEN_SIZE, HID), jnp.float32),
        ),
    )(embed_table, w_self, w_child)


def _scatter_body(tok_hbm, par_hbm, tch_hbm, c2_hbm,
                  par_res, tok_res, mpacked, pidx2, tok2, rows2,
                  chunk_acc, sem0, sem1):
    c = lax.axis_index("c")
    s = lax.axis_index("s")
    pltpu.sync_copy(par_hbm.at[pl.ds(s * TPT, TPT)], par_res)
    pltpu.sync_copy(tok_hbm.at[pl.ds(s * TPT, TPT)], tok_res)

    zeros16 = jnp.zeros((16,), jnp.float32)
    dumpv = jnp.full((16,), DUMPP, jnp.int32)
    lane16 = lax.iota(jnp.int32, 16)
    sems = (sem0, sem1)

    def chunk_body(k, carry):
        base = k * CH

        def zb(r, carry2):
            for q in range(8):
                rows2[0, r, pl.ds(q * 16, 16)] = zeros16
            return carry2

        lax.fori_loop(0, MB, zb, 0)
        pltpu.sync_copy(rows2.at[0], chunk_acc.at[pl.ds(s * 256, 128)])
        pltpu.sync_copy(rows2.at[0], chunk_acc.at[pl.ds(s * 256 + 128, 128)])

        @pl.when(s == 0)
        def _():
            pltpu.sync_copy(rows2.at[0].at[pl.ds(0, 8)],
                            chunk_acc.at[pl.ds(CH, 8)])

        plsc.subcore_barrier()

        def scan_body(i, off):
            pv = par_res[pl.ds(i * 16, 16)]
            lv = pv - base
            m = (lv >= 0) & (lv < CH)
            tv = tok_res[pl.ds(i * 16, 16)]
            packed = jnp.where(m, lv * 1024 + tv, DUMPP)
            spacked = lax.sort(packed)
            plsc.store_scatter(mpacked, [off + lane16], spacked)
            cnt = plsc.all_reduce_population_count(m)
            return off + cnt[0]

        n_m = lax.fori_loop(0, SCAN_IT, scan_body, jnp.int32(0))

        def padb(j, carry2):
            idx = n_m + j * 16 + lane16
            plsc.store_scatter(mpacked, [idx], dumpv)
            return carry2

        lax.fori_loop(0, MB // 16, padb, 0)

        nb = (n_m + (MB - 1)) // MB

        def stage(b, i):
            for q in range(MB // 16):
                pk = mpacked[pl.ds(b * MB + q * 16, 16)]
                pidx2[i, pl.ds(q * 16, 16)] = lax.shift_right_logical(pk, 10)
                tok2[i, pl.ds(q * 16, 16)] = jnp.bitwise_and(pk, 1023)
            pltpu.async_copy(tch_hbm.at[tok2.at[i]], rows2.at[i], sems[i])

        def drain(i):
            pltpu.make_async_copy(tch_hbm.at[tok2.at[i]], rows2.at[i],
                                  sems[i]).wait()
            pltpu.sync_copy(rows2.at[i], chunk_acc.at[pidx2.at[i]], add=True)

        @pl.when(nb > 0)
        def _():
            stage(0, 0)

        def bat2(p, carry2):
            b0 = 2 * p

            @pl.when(b0 + 1 < nb)
            def _():
                stage(b0 + 1, 1)

            drain(0)

            @pl.when(b0 + 2 < nb)
            def _():
                stage(b0 + 2, 0)

            @pl.when(b0 + 1 < nb)
            def _():
                drain(1)

            return carry2

        lax.fori_loop(0, (nb + 1) // 2, bat2, 0)
        plsc.subcore_barrier()

        pltpu.sync_copy(chunk_acc.at[pl.ds(s * 256, 128)],
                        c2_hbm.at[pl.ds(base + s * 256, 128)])
        pltpu.sync_copy(chunk_acc.at[pl.ds(s * 256 + 128, 128)],
                        c2_hbm.at[pl.ds(base + s * 256 + 128, 128)])
        return carry

    lo = c * SC0_CHUNKS
    hi = SC0_CHUNKS + c * (N_CHUNK - SC0_CHUNKS)
    lax.fori_loop(lo, hi, chunk_body, 0)


def _scatter(tok_pad, par_pad, tch):
    f = functools.partial(
        pl.kernel,
        out_type=jax.ShapeDtypeStruct((PAD_N, HID), jnp.float32),
        mesh=_mesh(),
        compiler_params=_SC_PARAMS,
        scratch_types=[
            pltpu.VMEM((TPT,), jnp.int32),
            pltpu.VMEM((TPT,), jnp.int32),
            pltpu.VMEM((TPT + MB,), jnp.int32),
            pltpu.VMEM((2, MB), jnp.int32),
            pltpu.VMEM((2, MB), jnp.int32),
            pltpu.VMEM((2, MB, HID), jnp.float32),
            pltpu.VMEM_SHARED((CHUNK_ROWS, HID), jnp.float32),
            pltpu.SemaphoreType.DMA,
            pltpu.SemaphoreType.DMA,
        ],
    )(_scatter_body)
    return f(tok_pad, par_pad, tch)


def _conv_body(tok_hbm, blk_hbm, ts_hbm, c2_hbm, bconv_hbm, out_hbm,
               tokb, blkb, ts_rows, c2_rows, bcv, blk_acc, sem):
    c = lax.axis_index("c")
    s = lax.axis_index("s")
    pltpu.sync_copy(bconv_hbm, bcv)

    zeros16 = jnp.zeros((16,), jnp.float32)

    def zb(r, carry):
        for q in range(8):
            ts_rows[r, pl.ds(q * 16, 16)] = zeros16
        return carry

    lax.fori_loop(0, 128, zb, 0)
    row0 = s * 632
    for j in range(4):
        pltpu.sync_copy(ts_rows, blk_acc.at[pl.ds(row0 + j * 128, 128)])
    pltpu.sync_copy(ts_rows.at[pl.ds(0, 120)],
                    blk_acc.at[pl.ds(row0 + 512, 120)])
    plsc.subcore_barrier()

    tbase = (c * 16 + s) * NPT

    def bat(b, carry):
        nstart = tbase + b * 128
        pltpu.sync_copy(tok_hbm.at[pl.ds(nstart, 128)], tokb)
        pltpu.sync_copy(blk_hbm.at[pl.ds(nstart, 128)], blkb)
        pltpu.async_copy(ts_hbm.at[tokb], ts_rows, sem).wait()
        pltpu.sync_copy(c2_hbm.at[pl.ds(nstart, 128)], c2_rows)

        def relu_b(r, carry2):
            for q in range(8):
                sl = pl.ds(q * 16, 16)
                ts_rows[r, sl] = jnp.maximum(
                    ts_rows[r, sl] + c2_rows[r, sl] + bcv[sl], 0.0)
            return carry2

        lax.fori_loop(0, 128, relu_b, 0)
        pltpu.sync_copy(ts_rows, blk_acc.at[blkb], add=True)
        return carry

    lax.fori_loop(0, NBAT, bat, 0)
    plsc.subcore_barrier()

    for j in range(4):
        pltpu.sync_copy(blk_acc.at[pl.ds(row0 + j * 128, 128)],
                        out_hbm.at[c].at[pl.ds(row0 + j * 128, 128)])
    pltpu.sync_copy(blk_acc.at[pl.ds(row0 + 512, 120)],
                    out_hbm.at[c].at[pl.ds(row0 + 512, 120)])


def _conv(tok_pad, blk_pad, ts, c2, b_conv):
    f = functools.partial(
        pl.kernel,
        out_type=jax.ShapeDtypeStruct((2, BLK_ROWS, HID), jnp.float32),
        mesh=_mesh(),
        compiler_params=_SC_PARAMS,
        scratch_types=[
            pltpu.VMEM((128,), jnp.int32),
            pltpu.VMEM((128,), jnp.int32),
            pltpu.VMEM((128, HID), jnp.float32),
            pltpu.VMEM((128, HID), jnp.float32),
            pltpu.VMEM((HID,), jnp.float32),
            pltpu.VMEM_SHARED((BLK_ROWS, HID), jnp.float32),
            pltpu.SemaphoreType.DMA,
        ],
    )(_conv_body)
    return f(tok_pad, blk_pad, ts, c2, b_conv)


def _block_body(acc_ref, wb_ref, bb_ref, wgi_ref, wgo_ref,
                asgi_ref, adgi_ref, asgo_ref, adgo_ref,
                bf_ref, hgi_aug_ref, hgo_aug_ref,
                o_asgi, o_adgi, o_asgo, o_adgo):
    bs = acc_ref[0] + acc_ref[1]
    bf = jnp.maximum(jnp.dot(bs, wb_ref[...], precision=PREC) + bb_ref[...],
                     0.0)
    bf_ref[...] = bf
    hgi = jnp.dot(bf, wgi_ref[...], precision=PREC)
    hgo = jnp.dot(bf, wgo_ref[...], precision=PREC)
    z = jnp.zeros((bf.shape[0], AUGW - HID // 2), jnp.float32)
    hgi_aug_ref[0] = jnp.concatenate([hgi[:, :64], z], axis=1)
    hgi_aug_ref[1] = jnp.concatenate([hgi[:, 64:], z], axis=1)
    hgo_aug_ref[0] = jnp.concatenate([hgo[:, :64], z], axis=1)
    hgo_aug_ref[1] = jnp.concatenate([hgo[:, 64:], z], axis=1)
    for out_ref, h, a_ref in ((o_asgi, hgi, asgi_ref), (o_adgi, hgi, adgi_ref),
                              (o_asgo, hgo, asgo_ref), (o_adgo, hgo, adgo_ref)):
        out_ref[0] = jnp.dot(h, a_ref[:, 0:2], precision=PREC)
        out_ref[1] = jnp.dot(h, a_ref[:, 2:4], precision=PREC)


def _block(acc, w_block, b_block, w_gi, w_go, a_sgi, a_dgi, a_sgo, a_dgo):
    R = 1000
    grid = (N_BLK // R,)
    full128 = pl.BlockSpec((HID, HID), lambda i: (0, 0))
    alph = pl.BlockSpec((HID, HEADS), lambda i: (0, 0))
    alph_out = pl.BlockSpec((2, R, 2), lambda i: (0, i, 0))
    aug_out = pl.BlockSpec((2, R, AUGW), lambda i: (0, i, 0))
    return pl.pallas_call(
        _block_body,
        grid=grid,
        in_specs=[
            pl.BlockSpec((2, R, HID), lambda i: (0, i, 0)),
            full128,
            pl.BlockSpec((1, HID), lambda i: (0, 0)),
            full128, full128,
            alph, alph, alph, alph,
        ],
        out_specs=[
            pl.BlockSpec((R, HID), lambda i: (i, 0)),
            aug_out, aug_out,
            alph_out, alph_out, alph_out, alph_out,
        ],
        out_shape=(
            jax.ShapeDtypeStruct((N_BLK, HID), jnp.float32),
            jax.ShapeDtypeStruct((2, N_BLK, AUGW), jnp.float32),
            jax.ShapeDtypeStruct((2, N_BLK, AUGW), jnp.float32),
            jax.ShapeDtypeStruct((2, N_BLK, 2), jnp.float32),
            jax.ShapeDtypeStruct((2, N_BLK, 2), jnp.float32),
            jax.ShapeDtypeStruct((2, N_BLK, 2), jnp.float32),
            jax.ShapeDtypeStruct((2, N_BLK, 2), jnp.float32),
        ),
    )(acc, w_block, b_block, w_gi, w_go, a_sgi, a_dgi, a_sgo, a_dgo)


def _gat_body(src_hbm, dst_hbm, et_hbm, asrc_hbm, adst_hbm, etb_hbm, haug_hbm,
              out_hbm, asrc_res, adst_res, etb_res, sb2, db2, tb2, sidx2,
              wb02, wb12, rows2, out_acc, sem0, sem1):
    c = lax.axis_index("c")
    s = lax.axis_index("s")
    pltpu.sync_copy(asrc_hbm.at[c], asrc_res)
    pltpu.sync_copy(adst_hbm.at[c], adst_res)
    pltpu.sync_copy(etb_hbm.at[c], etb_res)
    himask = jnp.full((16,), -65536, jnp.int32)

    zeros16 = jnp.zeros((16,), jnp.float32)

    def zb(r, carry):
        for q in range(AUGW // 16):
            rows2[0, r, pl.ds(q * 16, 16)] = zeros16
        return carry

    lax.fori_loop(0, EB, zb, 0)
    row0 = s * 632
    for j in range(8):
        pltpu.sync_copy(rows2.at[0].at[pl.ds(0, 79)],
                        out_acc.at[pl.ds(row0 + j * 79, 79)])
    plsc.subcore_barrier()

    coff = c * N_BLK
    lane = lax.iota(jnp.int32, 16)
    sems = (sem0, sem1)

    def stage(b, i):
        ebase = s * EPT + b * EB
        pltpu.sync_copy(src_hbm.at[pl.ds(ebase, EB)], sb2.at[i])
        pltpu.sync_copy(dst_hbm.at[pl.ds(ebase, EB)], db2.at[i])
        pltpu.sync_copy(et_hbm.at[pl.ds(ebase, EB)], tb2.at[i])
        for k in range(EB // 16):
            sl = pl.ds(k * 16, 16)
            sv = sb2[i, sl]
            dv = db2[i, sl]
            tv = tb2[i, sl]
            p1 = plsc.load_gather(asrc_res, [sv])
            p2 = plsc.load_gather(adst_res, [dv])
            p3 = plsc.load_gather(etb_res, [tv])
            for h in range(2):
                if h == 0:
                    a1 = plsc.bitcast(jnp.bitwise_and(p1, himask), jnp.float32)
                    a2 = plsc.bitcast(jnp.bitwise_and(p2, himask), jnp.float32)
                    a3 = plsc.bitcast(jnp.bitwise_and(p3, himask), jnp.float32)
                else:
                    a1 = plsc.bitcast(lax.shift_left(p1, 16), jnp.float32)
                    a2 = plsc.bitcast(lax.shift_left(p2, 16), jnp.float32)
                    a3 = plsc.bitcast(lax.shift_left(p3, 16), jnp.float32)
                e = a1 + a2 + a3
                e = jnp.where(e >= 0.0, e, 0.2 * e)
                w = jnp.exp(e)
                if h == 0:
                    wb02[i, sl] = w
                else:
                    wb12[i, sl] = w
            sidx2[i, sl] = sv + coff
        pltpu.async_copy(haug_hbm.at[sidx2.at[i]], rows2.at[i], sems[i])

    def drain(i):
        pltpu.make_async_copy(haug_hbm.at[sidx2.at[i]], rows2.at[i],
                              sems[i]).wait()

        def scale(k, carry2):
            wv0 = wb02[i, pl.ds(k * 16, 16)]
            wv1 = wb12[i, pl.ds(k * 16, 16)]
            for j16 in range(16):
                j = k * 16 + j16
                w0 = wv0[j16]
                w1 = wv1[j16]
                for q in range(2):
                    sl = pl.ds(q * 16, 16)
                    rows2[i, j, sl] = rows2[i, j, sl] * w0
                for q in range(2, 4):
                    sl = pl.ds(q * 16, 16)
                    rows2[i, j, sl] = rows2[i, j, sl] * w1
                rows2[i, j, pl.ds(64, 16)] = jnp.where(
                    lane == 0, w0, jnp.where(lane == 1, w1, 0.0))
            return carry2

        lax.fori_loop(0, EB // 16, scale, 0)
        pltpu.sync_copy(rows2.at[i], out_acc.at[db2.at[i]], add=True)

    stage(0, 0)
    NPAIR = NEB // 2

    def bat2(p, carry):
        stage(2 * p + 1, 1)
        drain(0)

        @pl.when(p < NPAIR - 1)
        def _():
            stage(2 * p + 2, 0)

        drain(1)
        return carry

    lax.fori_loop(0, NPAIR, bat2, 0)
    plsc.subcore_barrier()

    for j in range(4):
        pltpu.sync_copy(out_acc.at[pl.ds(row0 + j * 128, 128)],
                        out_hbm.at[c].at[pl.ds(row0 + j * 128, 128)])
    pltpu.sync_copy(out_acc.at[pl.ds(row0 + 512, 120)],
                    out_hbm.at[c].at[pl.ds(row0 + 512, 120)])


def _gat(srcx, dstx, etx, asrc_sc, adst_sc, etb_sc, haug):
    f = functools.partial(
        pl.kernel,
        out_type=jax.ShapeDtypeStruct((2, BLK_ROWS, AUGW), jnp.float32),
        mesh=_mesh(),
        compiler_params=_SC_PARAMS,
        scratch_types=[
            pltpu.VMEM((N_BLK,), jnp.int32),
            pltpu.VMEM((N_BLK,), jnp.int32),
            pltpu.VMEM((8,), jnp.int32),
            pltpu.VMEM((2, EB), jnp.int32),
            pltpu.VMEM((2, EB), jnp.int32),
            pltpu.VMEM((2, EB), jnp.int32),
            pltpu.VMEM((2, EB), jnp.int32),
            pltpu.VMEM((2, EB), jnp.float32),
            pltpu.VMEM((2, EB), jnp.float32),
            pltpu.VMEM((2, EB, AUGW), jnp.float32),
            pltpu.VMEM_SHARED((BLK_ROWS, AUGW), jnp.float32),
            pltpu.SemaphoreType.DMA,
            pltpu.SemaphoreType.DMA,
        ],
    )(_gat_body)
    return f(srcx, dstx, etx, asrc_sc, adst_sc, etb_sc, haug)


def _final_body(bf_ref, ain_ref, aout_ref, wm_ref, bm_ref, lab_ref,
                sig_ref, loss_ref, pooled_ref, macc):
    i = pl.program_id(0)

    @pl.when(i == 0)
    def _():
        macc[...] = jnp.full((1, HID), -jnp.inf, jnp.float32)

    parts = []
    for c in range(2):
        for g in range(2):
            num_i = ain_ref[c, :, g * DH:(g + 1) * DH]
            den_i = ain_ref[c, :, 64 + g:65 + g]
            num_o = aout_ref[c, :, g * DH:(g + 1) * DH]
            den_o = aout_ref[c, :, 64 + g:65 + g]
            parts.append(num_i / (den_i + 1e-9) + num_o / (den_o + 1e-9))
    fea = bf_ref[...] + jnp.concatenate(parts, axis=1)
    macc[...] = jnp.maximum(macc[...], jnp.max(fea, axis=0, keepdims=True))

    @pl.when(i == pl.num_programs(0) - 1)
    def _():
        pooled = macc[...]
        logits = jnp.dot(pooled, wm_ref[...], precision=PREC) + bm_ref[...]
        pooled_ref[...] = pooled
        sig_ref[...] = 1.0 / (1.0 + jnp.exp(-logits))
        l = logits[0, 0]
        y = lab_ref[0, 0]
        loss_ref[...] = (jnp.maximum(l, 0.0) - l * y +
                         jnp.log1p(jnp.exp(-jnp.abs(l)))).reshape(1, 1)


def _final(bf, accin, accout, w_mlp, b_mlp, lab):
    R = 1000
    grid = (N_BLK // R,)
    return pl.pallas_call(
        _final_body,
        grid=grid,
        in_specs=[
            pl.BlockSpec((R, HID), lambda i: (i, 0)),
            pl.BlockSpec((2, R, AUGW), lambda i: (0, i, 0)),
            pl.BlockSpec((2, R, AUGW), lambda i: (0, i, 0)),
            pl.BlockSpec((HID, 1), lambda i: (0, 0)),
            pl.BlockSpec((1, 1), lambda i: (0, 0)),
            pl.BlockSpec((1, 1), lambda i: (0, 0)),
        ],
        out_specs=[
            pl.BlockSpec((1, 1), lambda i: (0, 0)),
            pl.BlockSpec((1, 1), lambda i: (0, 0)),
            pl.BlockSpec((1, HID), lambda i: (0, 0)),
        ],
        out_shape=(
            jax.ShapeDtypeStruct((1, 1), jnp.float32),
            jax.ShapeDtypeStruct((1, 1), jnp.float32),
            jax.ShapeDtypeStruct((1, HID), jnp.float32),
        ),
        scratch_shapes=[pltpu.VMEM((1, HID), jnp.float32)],
    )(bf, accin, accout, w_mlp, b_mlp, lab)


def _pack_pair(a):
    bb = lax.bitcast_convert_type(a.astype(jnp.bfloat16),
                                  jnp.uint16).astype(jnp.uint32)
    return (jnp.left_shift(bb[..., 0], 16) | bb[..., 1]).astype(jnp.int32)


def _alpha_mat(a):
    m = jnp.zeros((HID, HEADS), jnp.float32)
    for h in range(HEADS):
        m = m.at[h * DH:(h + 1) * DH, h].set(a[h])
    return m


def kernel(ast_tokens, ast_parent, emb_block_id, cfg_edge_index, cfg_edge_type,
           label, embed_table, W_self, W_child, b_conv, W_block, b_block,
           W_gi, a_src_gi, a_dst_gi, et_gi,
           W_go, a_src_go, a_dst_go, et_go,
           W_mlp, b_mlp):
    i32 = jnp.int32
    npad = PAD_N - N_AST
    tok_pad = jnp.concatenate([ast_tokens.astype(i32),
                               jnp.zeros((npad,), i32)])
    par_pad = jnp.concatenate([ast_parent.astype(i32),
                               jnp.full((npad,), -1, i32)])
    blk_pad = jnp.concatenate([emb_block_id.astype(i32),
                               jnp.full((npad,), N_BLK, i32)])

    ts, tch = _prep(embed_table, W_self, W_child)
    c2 = _scatter(tok_pad, par_pad, tch)
    acc = _conv(tok_pad, blk_pad, ts, c2, b_conv)

    bf, haug_gi, haug_go, asgi, adgi, asgo, adgo = _block(
        acc, W_block, b_block.reshape(1, HID), W_gi, W_go,
        _alpha_mat(a_src_gi), _alpha_mat(a_dst_gi),
        _alpha_mat(a_src_go), _alpha_mat(a_dst_go))

    srcx = cfg_edge_index[0].astype(i32)
    dstx = cfg_edge_index[1].astype(i32)
    etx = cfg_edge_type.astype(i32)
    asgi = _pack_pair(asgi)
    adgi = _pack_pair(adgi)
    asgo = _pack_pair(asgo)
    adgo = _pack_pair(adgo)
    etb_gi = _pack_pair(jnp.pad(
        et_gi.reshape(3, 2, 2).transpose(1, 0, 2), ((0, 0), (0, 5), (0, 0))))
    etb_go = _pack_pair(jnp.pad(
        et_go.reshape(3, 2, 2).transpose(1, 0, 2), ((0, 0), (0, 5), (0, 0))))

    accin = _gat(srcx, dstx, etx, asgi, adgi, etb_gi,
                 haug_gi.reshape(2 * N_BLK, AUGW))
    accout = _gat(dstx, srcx, etx, asgo, adgo, etb_go,
                  haug_go.reshape(2 * N_BLK, AUGW))

    sig, loss, pooled = _final(bf, accin, accout, W_mlp,
                               b_mlp.reshape(1, 1), label.reshape(1, 1))
    return (sig, loss.reshape(()), pooled)


# speedup vs baseline: 30.0940x; 1.0991x over previous
import functools

import jax
import jax.numpy as jnp
from jax import lax
from jax.experimental import pallas as pl
from jax.experimental.pallas import tpu as pltpu
from jax.experimental.pallas import tpu_sc as plsc

N_AST = 100000
N_BLK = 10000
E_CFG = 320000
TOKEN_SIZE = 1000
FEAT = 128
HID = 128
HEADS = 4
DH = HID // HEADS

PAD_N = 106496
N_CHUNK = 13
CH = 8192
SC0_CHUNKS = 7
CHUNK_ROWS = CH + 8
BLK_ROWS = 10112
TPT = PAD_N // 16
SCAN_IT = TPT // 16
MB = 128
NPT = PAD_N // 32
NBAT = NPT // 128
EPT = E_CFG // 16
EB = 80
NEB = EPT // EB
AUGW = 128
DUMPP = CH * 1024
PREC = lax.Precision.HIGHEST


def _mesh():
    return plsc.VectorSubcoreMesh(core_axis_name="c", subcore_axis_name="s",
                                  num_cores=2, num_subcores=16)


_SC_PARAMS = pltpu.CompilerParams(needs_layout_passes=False)


def _prep_body(emb_ref, ws_ref, wc_ref, ts_ref, tch_ref):
    ts_ref[...] = jnp.dot(emb_ref[...], ws_ref[...], precision=PREC)
    tch_ref[...] = jnp.dot(emb_ref[...], wc_ref[...], precision=PREC)


def _prep(embed_table, w_self, w_child):
    return pl.pallas_call(
        _prep_body,
        out_shape=(
            jax.ShapeDtypeStruct((TOKEN_SIZE, HID), jnp.float32),
            jax.ShapeDtypeStruct((TOKEN_SIZE, HID), jnp.float32),
        ),
    )(embed_table, w_self, w_child)


def _scatter_body(tok_hbm, par_hbm, tch_hbm, c2_hbm,
                  par_res, tok_res, mpacked, pidx2, tok2, rows2,
                  chunk_acc, sem0, sem1):
    c = lax.axis_index("c")
    s = lax.axis_index("s")
    pltpu.sync_copy(par_hbm.at[pl.ds(s * TPT, TPT)], par_res)
    pltpu.sync_copy(tok_hbm.at[pl.ds(s * TPT, TPT)], tok_res)

    zeros16 = jnp.zeros((16,), jnp.float32)
    dumpv = jnp.full((16,), DUMPP, jnp.int32)
    lane16 = lax.iota(jnp.int32, 16)
    sems = (sem0, sem1)

    def chunk_body(k, carry):
        base = k * CH

        def zb(r, carry2):
            for q in range(8):
                rows2[0, r, pl.ds(q * 16, 16)] = zeros16
            return carry2

        lax.fori_loop(0, MB, zb, 0)
        for j in range(4):
            pltpu.sync_copy(rows2.at[0],
                            chunk_acc.at[pl.ds(s * 512 + j * 128, 128)])

        @pl.when(s == 0)
        def _():
            pltpu.sync_copy(rows2.at[0].at[pl.ds(0, 8)],
                            chunk_acc.at[pl.ds(CH, 8)])

        plsc.subcore_barrier()

        def scan_body(i, off):
            pv = par_res[pl.ds(i * 16, 16)]
            lv = pv - base
            m = (lv >= 0) & (lv < CH)
            tv = tok_res[pl.ds(i * 16, 16)]
            packed = jnp.where(m, lv * 1024 + tv, DUMPP)
            spacked = lax.sort(packed)
            plsc.store_scatter(mpacked, [off + lane16], spacked)
            cnt = plsc.all_reduce_population_count(m)
            return off + cnt[0]

        n_m = lax.fori_loop(0, SCAN_IT, scan_body, jnp.int32(0))

        def padb(j, carry2):
            idx = n_m + j * 16 + lane16
            plsc.store_scatter(mpacked, [idx], dumpv)
            return carry2

        lax.fori_loop(0, MB // 16, padb, 0)

        nb = (n_m + (MB - 1)) // MB

        def stage(b, i):
            for q in range(MB // 16):
                pk = mpacked[pl.ds(b * MB + q * 16, 16)]
                pidx2[i, pl.ds(q * 16, 16)] = lax.shift_right_logical(pk, 10)
                tok2[i, pl.ds(q * 16, 16)] = jnp.bitwise_and(pk, 1023)
            pltpu.async_copy(tch_hbm.at[tok2.at[i]], rows2.at[i], sems[i])

        def drain(i):
            pltpu.make_async_copy(tch_hbm.at[tok2.at[i]], rows2.at[i],
                                  sems[i]).wait()
            pltpu.sync_copy(rows2.at[i], chunk_acc.at[pidx2.at[i]], add=True)

        @pl.when(nb > 0)
        def _():
            stage(0, 0)

        def bat2(p, carry2):
            b0 = 2 * p

            @pl.when(b0 + 1 < nb)
            def _():
                stage(b0 + 1, 1)

            drain(0)

            @pl.when(b0 + 2 < nb)
            def _():
                stage(b0 + 2, 0)

            @pl.when(b0 + 1 < nb)
            def _():
                drain(1)

            return carry2

        lax.fori_loop(0, (nb + 1) // 2, bat2, 0)
        plsc.subcore_barrier()

        for j in range(4):
            pltpu.sync_copy(chunk_acc.at[pl.ds(s * 512 + j * 128, 128)],
                            c2_hbm.at[pl.ds(base + s * 512 + j * 128, 128)])
        return carry

    lo = c * SC0_CHUNKS
    hi = SC0_CHUNKS + c * (N_CHUNK - SC0_CHUNKS)
    lax.fori_loop(lo, hi, chunk_body, 0)


def _scatter(tok_pad, par_pad, tch):
    f = functools.partial(
        pl.kernel,
        out_type=jax.ShapeDtypeStruct((PAD_N, HID), jnp.float32),
        mesh=_mesh(),
        compiler_params=_SC_PARAMS,
        scratch_types=[
            pltpu.VMEM((TPT,), jnp.int32),
            pltpu.VMEM((TPT,), jnp.int32),
            pltpu.VMEM((TPT + MB,), jnp.int32),
            pltpu.VMEM((2, MB), jnp.int32),
            pltpu.VMEM((2, MB), jnp.int32),
            pltpu.VMEM((2, MB, HID), jnp.float32),
            pltpu.VMEM_SHARED((CHUNK_ROWS, HID), jnp.float32),
            pltpu.SemaphoreType.DMA,
            pltpu.SemaphoreType.DMA,
        ],
    )(_scatter_body)
    return f(tok_pad, par_pad, tch)


def _conv_body(tok_hbm, blk_hbm, ts_hbm, c2_hbm, bconv_hbm, out_hbm,
               tokb, blkb, ts_rows, c2_rows, bcv, blk_acc, sem):
    c = lax.axis_index("c")
    s = lax.axis_index("s")
    pltpu.sync_copy(bconv_hbm, bcv)

    zeros16 = jnp.zeros((16,), jnp.float32)

    def zb(r, carry):
        for q in range(8):
            ts_rows[r, pl.ds(q * 16, 16)] = zeros16
        return carry

    lax.fori_loop(0, 128, zb, 0)
    row0 = s * 632
    for j in range(4):
        pltpu.sync_copy(ts_rows, blk_acc.at[pl.ds(row0 + j * 128, 128)])
    pltpu.sync_copy(ts_rows.at[pl.ds(0, 120)],
                    blk_acc.at[pl.ds(row0 + 512, 120)])
    plsc.subcore_barrier()

    tbase = (c * 16 + s) * NPT

    def bat(b, carry):
        nstart = tbase + b * 128
        pltpu.sync_copy(tok_hbm.at[pl.ds(nstart, 128)], tokb)
        pltpu.sync_copy(blk_hbm.at[pl.ds(nstart, 128)], blkb)
        pltpu.async_copy(ts_hbm.at[tokb], ts_rows, sem).wait()
        pltpu.sync_copy(c2_hbm.at[pl.ds(nstart, 128)], c2_rows)

        def relu_b(r, carry2):
            for q in range(8):
                sl = pl.ds(q * 16, 16)
                ts_rows[r, sl] = jnp.maximum(
                    ts_rows[r, sl] + c2_rows[r, sl] + bcv[sl], 0.0)
            return carry2

        lax.fori_loop(0, 128, relu_b, 0)
        pltpu.sync_copy(ts_rows, blk_acc.at[blkb], add=True)
        return carry

    lax.fori_loop(0, NBAT, bat, 0)
    plsc.subcore_barrier()

    for j in range(4):
        pltpu.sync_copy(blk_acc.at[pl.ds(row0 + j * 128, 128)],
                        out_hbm.at[c].at[pl.ds(row0 + j * 128, 128)])
    pltpu.sync_copy(blk_acc.at[pl.ds(row0 + 512, 120)],
                    out_hbm.at[c].at[pl.ds(row0 + 512, 120)])


def _conv(tok_pad, blk_pad, ts, c2, b_conv):
    f = functools.partial(
        pl.kernel,
        out_type=jax.ShapeDtypeStruct((2, BLK_ROWS, HID), jnp.float32),
        mesh=_mesh(),
        compiler_params=_SC_PARAMS,
        scratch_types=[
            pltpu.VMEM((128,), jnp.int32),
            pltpu.VMEM((128,), jnp.int32),
            pltpu.VMEM((128, HID), jnp.float32),
            pltpu.VMEM((128, HID), jnp.float32),
            pltpu.VMEM((HID,), jnp.float32),
            pltpu.VMEM_SHARED((BLK_ROWS, HID), jnp.float32),
            pltpu.SemaphoreType.DMA,
        ],
    )(_conv_body)
    return f(tok_pad, blk_pad, ts, c2, b_conv)


def _block_body(acc_ref, wb_ref, bb_ref, wgi_ref, wgo_ref,
                asgi_ref, adgi_ref, asgo_ref, adgo_ref,
                bf_ref, hgi_aug_ref, hgo_aug_ref,
                o_asgi, o_adgi, o_asgo, o_adgo):
    bs = acc_ref[0] + acc_ref[1]
    bf = jnp.maximum(jnp.dot(bs, wb_ref[...], precision=PREC) + bb_ref[...],
                     0.0)
    bf_ref[...] = bf
    hgi = jnp.dot(bf, wgi_ref[...], precision=PREC)
    hgo = jnp.dot(bf, wgo_ref[...], precision=PREC)
    z = jnp.zeros((bf.shape[0], AUGW - HID // 2), jnp.float32)
    hgi_aug_ref[0] = jnp.concatenate([hgi[:, :64], z], axis=1)
    hgi_aug_ref[1] = jnp.concatenate([hgi[:, 64:], z], axis=1)
    hgo_aug_ref[0] = jnp.concatenate([hgo[:, :64], z], axis=1)
    hgo_aug_ref[1] = jnp.concatenate([hgo[:, 64:], z], axis=1)
    for out_ref, h, a_ref in ((o_asgi, hgi, asgi_ref), (o_adgi, hgi, adgi_ref),
                              (o_asgo, hgo, asgo_ref), (o_adgo, hgo, adgo_ref)):
        out_ref[0] = jnp.dot(h, a_ref[:, 0:2], precision=PREC)
        out_ref[1] = jnp.dot(h, a_ref[:, 2:4], precision=PREC)


def _block(acc, w_block, b_block, w_gi, w_go, a_sgi, a_dgi, a_sgo, a_dgo):
    R = 1000
    grid = (N_BLK // R,)
    full128 = pl.BlockSpec((HID, HID), lambda i: (0, 0))
    alph = pl.BlockSpec((HID, HEADS), lambda i: (0, 0))
    alph_out = pl.BlockSpec((2, R, 2), lambda i: (0, i, 0))
    aug_out = pl.BlockSpec((2, R, AUGW), lambda i: (0, i, 0))
    return pl.pallas_call(
        _block_body,
        grid=grid,
        in_specs=[
            pl.BlockSpec((2, R, HID), lambda i: (0, i, 0)),
            full128,
            pl.BlockSpec((1, HID), lambda i: (0, 0)),
            full128, full128,
            alph, alph, alph, alph,
        ],
        out_specs=[
            pl.BlockSpec((R, HID), lambda i: (i, 0)),
            aug_out, aug_out,
            alph_out, alph_out, alph_out, alph_out,
        ],
        out_shape=(
            jax.ShapeDtypeStruct((N_BLK, HID), jnp.float32),
            jax.ShapeDtypeStruct((2, N_BLK, AUGW), jnp.float32),
            jax.ShapeDtypeStruct((2, N_BLK, AUGW), jnp.float32),
            jax.ShapeDtypeStruct((2, N_BLK, 2), jnp.float32),
            jax.ShapeDtypeStruct((2, N_BLK, 2), jnp.float32),
            jax.ShapeDtypeStruct((2, N_BLK, 2), jnp.float32),
            jax.ShapeDtypeStruct((2, N_BLK, 2), jnp.float32),
        ),
    )(acc, w_block, b_block, w_gi, w_go, a_sgi, a_dgi, a_sgo, a_dgo)


def _gat_body(src_hbm, dst_hbm, et_hbm, asrc_hbm, adst_hbm, etb_hbm, haug_hbm,
              out_hbm, asrc_res, adst_res, etb_res, sb2, db2, tb2, sidx2,
              wb02, wb12, rows2, out_acc, sem0, sem1):
    c = lax.axis_index("c")
    s = lax.axis_index("s")
    pltpu.sync_copy(asrc_hbm.at[c], asrc_res)
    pltpu.sync_copy(adst_hbm.at[c], adst_res)
    pltpu.sync_copy(etb_hbm.at[c], etb_res)
    himask = jnp.full((16,), -65536, jnp.int32)

    zeros16 = jnp.zeros((16,), jnp.float32)

    def zb(r, carry):
        for q in range(AUGW // 16):
            rows2[0, r, pl.ds(q * 16, 16)] = zeros16
        return carry

    lax.fori_loop(0, EB, zb, 0)
    row0 = s * 632
    for j in range(8):
        pltpu.sync_copy(rows2.at[0].at[pl.ds(0, 79)],
                        out_acc.at[pl.ds(row0 + j * 79, 79)])
    plsc.subcore_barrier()

    coff = c * N_BLK
    lane = lax.iota(jnp.int32, 16)
    sems = (sem0, sem1)

    def stage(b, i):
        ebase = s * EPT + b * EB
        pltpu.sync_copy(src_hbm.at[pl.ds(ebase, EB)], sb2.at[i])
        pltpu.sync_copy(dst_hbm.at[pl.ds(ebase, EB)], db2.at[i])
        pltpu.sync_copy(et_hbm.at[pl.ds(ebase, EB)], tb2.at[i])
        for k in range(EB // 16):
            sl = pl.ds(k * 16, 16)
            sv = sb2[i, sl]
            dv = db2[i, sl]
            tv = tb2[i, sl]
            p1 = plsc.load_gather(asrc_res, [sv])
            p2 = plsc.load_gather(adst_res, [dv])
            p3 = plsc.load_gather(etb_res, [tv])
            for h in range(2):
                if h == 0:
                    a1 = plsc.bitcast(jnp.bitwise_and(p1, himask), jnp.float32)
                    a2 = plsc.bitcast(jnp.bitwise_and(p2, himask), jnp.float32)
                    a3 = plsc.bitcast(jnp.bitwise_and(p3, himask), jnp.float32)
                else:
                    a1 = plsc.bitcast(lax.shift_left(p1, 16), jnp.float32)
                    a2 = plsc.bitcast(lax.shift_left(p2, 16), jnp.float32)
                    a3 = plsc.bitcast(lax.shift_left(p3, 16), jnp.float32)
                e = a1 + a2 + a3
                e = jnp.where(e >= 0.0, e, 0.2 * e)
                w = jnp.exp(e)
                if h == 0:
                    wb02[i, sl] = w
                else:
                    wb12[i, sl] = w
            sidx2[i, sl] = sv + coff
        pltpu.async_copy(haug_hbm.at[sidx2.at[i]], rows2.at[i], sems[i])

    def drain(i):
        pltpu.make_async_copy(haug_hbm.at[sidx2.at[i]], rows2.at[i],
                              sems[i]).wait()

        def scale(k, carry2):
            wv0 = wb02[i, pl.ds(k * 16, 16)]
            wv1 = wb12[i, pl.ds(k * 16, 16)]
            for j16 in range(16):
                j = k * 16 + j16
                w0 = wv0[j16]
                w1 = wv1[j16]
                for q in range(2):
                    sl = pl.ds(q * 16, 16)
                    rows2[i, j, sl] = rows2[i, j, sl] * w0
                for q in range(2, 4):
                    sl = pl.ds(q * 16, 16)
                    rows2[i, j, sl] = rows2[i, j, sl] * w1
                rows2[i, j, pl.ds(64, 16)] = jnp.where(
                    lane == 0, w0, jnp.where(lane == 1, w1, 0.0))
            return carry2

        lax.fori_loop(0, EB // 16, scale, 0)
        pltpu.sync_copy(rows2.at[i], out_acc.at[db2.at[i]], add=True)

    stage(0, 0)
    NPAIR = NEB // 2

    def bat2(p, carry):
        stage(2 * p + 1, 1)
        drain(0)

        @pl.when(p < NPAIR - 1)
        def _():
            stage(2 * p + 2, 0)

        drain(1)
        return carry

    lax.fori_loop(0, NPAIR, bat2, 0)
    plsc.subcore_barrier()

    for j in range(4):
        pltpu.sync_copy(out_acc.at[pl.ds(row0 + j * 128, 128)],
                        out_hbm.at[c].at[pl.ds(row0 + j * 128, 128)])
    pltpu.sync_copy(out_acc.at[pl.ds(row0 + 512, 120)],
                    out_hbm.at[c].at[pl.ds(row0 + 512, 120)])


def _gat(srcx, dstx, etx, asrc_sc, adst_sc, etb_sc, haug):
    f = functools.partial(
        pl.kernel,
        out_type=jax.ShapeDtypeStruct((2, BLK_ROWS, AUGW), jnp.float32),
        mesh=_mesh(),
        compiler_params=_SC_PARAMS,
        scratch_types=[
            pltpu.VMEM((N_BLK,), jnp.int32),
            pltpu.VMEM((N_BLK,), jnp.int32),
            pltpu.VMEM((8,), jnp.int32),
            pltpu.VMEM((2, EB), jnp.int32),
            pltpu.VMEM((2, EB), jnp.int32),
            pltpu.VMEM((2, EB), jnp.int32),
            pltpu.VMEM((2, EB), jnp.int32),
            pltpu.VMEM((2, EB), jnp.float32),
            pltpu.VMEM((2, EB), jnp.float32),
            pltpu.VMEM((2, EB, AUGW), jnp.float32),
            pltpu.VMEM_SHARED((BLK_ROWS, AUGW), jnp.float32),
            pltpu.SemaphoreType.DMA,
            pltpu.SemaphoreType.DMA,
        ],
    )(_gat_body)
    return f(srcx, dstx, etx, asrc_sc, adst_sc, etb_sc, haug)


def _final_body(bf_ref, ain_ref, aout_ref, wm_ref, bm_ref, lab_ref,
                sig_ref, loss_ref, pooled_ref, macc):
    i = pl.program_id(0)

    @pl.when(i == 0)
    def _():
        macc[...] = jnp.full((1, HID), -jnp.inf, jnp.float32)

    parts = []
    for c in range(2):
        for g in range(2):
            num_i = ain_ref[c, :, g * DH:(g + 1) * DH]
            den_i = ain_ref[c, :, 64 + g:65 + g]
            num_o = aout_ref[c, :, g * DH:(g + 1) * DH]
            den_o = aout_ref[c, :, 64 + g:65 + g]
            parts.append(num_i / (den_i + 1e-9) + num_o / (den_o + 1e-9))
    fea = bf_ref[...] + jnp.concatenate(parts, axis=1)
    macc[...] = jnp.maximum(macc[...], jnp.max(fea, axis=0, keepdims=True))

    @pl.when(i == pl.num_programs(0) - 1)
    def _():
        pooled = macc[...]
        logits = jnp.dot(pooled, wm_ref[...], precision=PREC) + bm_ref[...]
        pooled_ref[...] = pooled
        sig_ref[...] = 1.0 / (1.0 + jnp.exp(-logits))
        l = logits[0, 0]
        y = lab_ref[0, 0]
        loss_ref[...] = (jnp.maximum(l, 0.0) - l * y +
                         jnp.log1p(jnp.exp(-jnp.abs(l)))).reshape(1, 1)


def _final(bf, accin, accout, w_mlp, b_mlp, lab):
    R = 1000
    grid = (N_BLK // R,)
    return pl.pallas_call(
        _final_body,
        grid=grid,
        in_specs=[
            pl.BlockSpec((R, HID), lambda i: (i, 0)),
            pl.BlockSpec((2, R, AUGW), lambda i: (0, i, 0)),
            pl.BlockSpec((2, R, AUGW), lambda i: (0, i, 0)),
            pl.BlockSpec((HID, 1), lambda i: (0, 0)),
            pl.BlockSpec((1, 1), lambda i: (0, 0)),
            pl.BlockSpec((1, 1), lambda i: (0, 0)),
        ],
        out_specs=[
            pl.BlockSpec((1, 1), lambda i: (0, 0)),
            pl.BlockSpec((1, 1), lambda i: (0, 0)),
            pl.BlockSpec((1, HID), lambda i: (0, 0)),
        ],
        out_shape=(
            jax.ShapeDtypeStruct((1, 1), jnp.float32),
            jax.ShapeDtypeStruct((1, 1), jnp.float32),
            jax.ShapeDtypeStruct((1, HID), jnp.float32),
        ),
        scratch_shapes=[pltpu.VMEM((1, HID), jnp.float32)],
    )(bf, accin, accout, w_mlp, b_mlp, lab)


def _pack_pair(a):
    bb = lax.bitcast_convert_type(a.astype(jnp.bfloat16),
                                  jnp.uint16).astype(jnp.uint32)
    return (jnp.left_shift(bb[..., 0], 16) | bb[..., 1]).astype(jnp.int32)


def _alpha_mat(a):
    m = jnp.zeros((HID, HEADS), jnp.float32)
    for h in range(HEADS):
        m = m.at[h * DH:(h + 1) * DH, h].set(a[h])
    return m


def kernel(ast_tokens, ast_parent, emb_block_id, cfg_edge_index, cfg_edge_type,
           label, embed_table, W_self, W_child, b_conv, W_block, b_block,
           W_gi, a_src_gi, a_dst_gi, et_gi,
           W_go, a_src_go, a_dst_go, et_go,
           W_mlp, b_mlp):
    i32 = jnp.int32
    npad = PAD_N - N_AST
    tok_pad = jnp.concatenate([ast_tokens.astype(i32),
                               jnp.zeros((npad,), i32)])
    par_pad = jnp.concatenate([ast_parent.astype(i32),
                               jnp.full((npad,), -1, i32)])
    blk_pad = jnp.concatenate([emb_block_id.astype(i32),
                               jnp.full((npad,), N_BLK, i32)])

    ts, tch = _prep(embed_table, W_self, W_child)
    c2 = _scatter(tok_pad, par_pad, tch)
    acc = _conv(tok_pad, blk_pad, ts, c2, b_conv)

    bf, haug_gi, haug_go, asgi, adgi, asgo, adgo = _block(
        acc, W_block, b_block.reshape(1, HID), W_gi, W_go,
        _alpha_mat(a_src_gi), _alpha_mat(a_dst_gi),
        _alpha_mat(a_src_go), _alpha_mat(a_dst_go))

    srcx = cfg_edge_index[0].astype(i32)
    dstx = cfg_edge_index[1].astype(i32)
    etx = cfg_edge_type.astype(i32)
    asgi = _pack_pair(asgi)
    adgi = _pack_pair(adgi)
    asgo = _pack_pair(asgo)
    adgo = _pack_pair(adgo)
    etb_gi = _pack_pair(jnp.pad(
        et_gi.reshape(3, 2, 2).transpose(1, 0, 2), ((0, 0), (0, 5), (0, 0))))
    etb_go = _pack_pair(jnp.pad(
        et_go.reshape(3, 2, 2).transpose(1, 0, 2), ((0, 0), (0, 5), (0, 0))))

    accin = _gat(srcx, dstx, etx, asgi, adgi, etb_gi,
                 haug_gi.reshape(2 * N_BLK, AUGW))
    accout = _gat(dstx, srcx, etx, asgo, adgo, etb_go,
                  haug_go.reshape(2 * N_BLK, AUGW))

    sig, loss, pooled = _final(bf, accin, accout, W_mlp,
                               b_mlp.reshape(1, 1), label.reshape(1, 1))
    return (sig, loss.reshape(()), pooled)


# speedup vs baseline: 30.4260x; 1.0110x over previous
import functools

import jax
import jax.numpy as jnp
from jax import lax
from jax.experimental import pallas as pl
from jax.experimental.pallas import tpu as pltpu
from jax.experimental.pallas import tpu_sc as plsc

N_AST = 100000
N_BLK = 10000
E_CFG = 320000
TOKEN_SIZE = 1000
FEAT = 128
HID = 128
HEADS = 4
DH = HID // HEADS

PAD_N = 106496
N_CHUNK = 13
CH = 8192
SC0_CHUNKS = 7
CHUNK_ROWS = CH + 8
BLK_ROWS = 10112
TPT = PAD_N // 16
SCAN_IT = TPT // 16
MB = 128
NPT = PAD_N // 32
NBAT = NPT // 128
EPT = E_CFG // 16
EB = 80
NEB = EPT // EB
AUGW = 128
DUMPP = CH * 1024
PREC = lax.Precision.HIGHEST


def _mesh():
    return plsc.VectorSubcoreMesh(core_axis_name="c", subcore_axis_name="s",
                                  num_cores=2, num_subcores=16)


_SC_PARAMS = pltpu.CompilerParams(needs_layout_passes=False)


def _prep_body(emb_ref, ws_ref, wc_ref, ts_ref, tch_ref):
    ts_ref[...] = jnp.dot(emb_ref[...], ws_ref[...], precision=PREC)
    tch_ref[...] = jnp.dot(emb_ref[...], wc_ref[...], precision=PREC)


def _prep(embed_table, w_self, w_child):
    return pl.pallas_call(
        _prep_body,
        out_shape=(
            jax.ShapeDtypeStruct((TOKEN_SIZE, HID), jnp.float32),
            jax.ShapeDtypeStruct((TOKEN_SIZE, HID), jnp.float32),
        ),
    )(embed_table, w_self, w_child)


def _scatter_body(tok_hbm, par_hbm, tch_hbm, c2_hbm,
                  par_res, tok_res, mpacked, pidx2, tok2, rows2,
                  chunk_acc, sem0, sem1):
    c = lax.axis_index("c")
    s = lax.axis_index("s")
    pltpu.sync_copy(par_hbm.at[pl.ds(s * TPT, TPT)], par_res)
    pltpu.sync_copy(tok_hbm.at[pl.ds(s * TPT, TPT)], tok_res)

    zeros16 = jnp.zeros((16,), jnp.float32)
    dumpv = jnp.full((16,), DUMPP, jnp.int32)
    lane16 = lax.iota(jnp.int32, 16)
    sems = (sem0, sem1)

    def chunk_body(k, carry):
        base = k * CH

        def zb(r, carry2):
            for q in range(8):
                rows2[0, r, pl.ds(q * 16, 16)] = zeros16
            return carry2

        lax.fori_loop(0, MB, zb, 0)
        for j in range(4):
            pltpu.sync_copy(rows2.at[0],
                            chunk_acc.at[pl.ds(s * 512 + j * 128, 128)])

        @pl.when(s == 0)
        def _():
            pltpu.sync_copy(rows2.at[0].at[pl.ds(0, 8)],
                            chunk_acc.at[pl.ds(CH, 8)])

        plsc.subcore_barrier()

        def scan_body(i, off):
            sp = []
            cn = []
            for u in range(4):
                pv = par_res[pl.ds(i * 64 + u * 16, 16)]
                lv = pv - base
                m = (lv >= 0) & (lv < CH)
                tv = tok_res[pl.ds(i * 64 + u * 16, 16)]
                packed = jnp.where(m, lv * 1024 + tv, DUMPP)
                sp.append(lax.sort(packed))
                cn.append(plsc.all_reduce_population_count(m)[0])
            for u in range(4):
                plsc.store_scatter(mpacked, [off + lane16], sp[u])
                off = off + cn[u]
            return off

        n_m = lax.fori_loop(0, SCAN_IT // 4, scan_body, jnp.int32(0))

        def padb(j, carry2):
            idx = n_m + j * 16 + lane16
            plsc.store_scatter(mpacked, [idx], dumpv)
            return carry2

        lax.fori_loop(0, MB // 16, padb, 0)

        nb = (n_m + (MB - 1)) // MB

        def stage(b, i):
            for q in range(MB // 16):
                pk = mpacked[pl.ds(b * MB + q * 16, 16)]
                pidx2[i, pl.ds(q * 16, 16)] = lax.shift_right_logical(pk, 10)
                tok2[i, pl.ds(q * 16, 16)] = jnp.bitwise_and(pk, 1023)
            pltpu.async_copy(tch_hbm.at[tok2.at[i]], rows2.at[i], sems[i])

        def drain(i):
            pltpu.make_async_copy(tch_hbm.at[tok2.at[i]], rows2.at[i],
                                  sems[i]).wait()
            pltpu.sync_copy(rows2.at[i], chunk_acc.at[pidx2.at[i]], add=True)

        @pl.when(nb > 0)
        def _():
            stage(0, 0)

        def bat2(p, carry2):
            b0 = 2 * p

            @pl.when(b0 + 1 < nb)
            def _():
                stage(b0 + 1, 1)

            drain(0)

            @pl.when(b0 + 2 < nb)
            def _():
                stage(b0 + 2, 0)

            @pl.when(b0 + 1 < nb)
            def _():
                drain(1)

            return carry2

        lax.fori_loop(0, (nb + 1) // 2, bat2, 0)
        plsc.subcore_barrier()

        for j in range(4):
            pltpu.sync_copy(chunk_acc.at[pl.ds(s * 512 + j * 128, 128)],
                            c2_hbm.at[pl.ds(base + s * 512 + j * 128, 128)])
        return carry

    lo = c * SC0_CHUNKS
    hi = SC0_CHUNKS + c * (N_CHUNK - SC0_CHUNKS)
    lax.fori_loop(lo, hi, chunk_body, 0)


def _scatter(tok_pad, par_pad, tch):
    f = functools.partial(
        pl.kernel,
        out_type=jax.ShapeDtypeStruct((PAD_N, HID), jnp.float32),
        mesh=_mesh(),
        compiler_params=_SC_PARAMS,
        scratch_types=[
            pltpu.VMEM((TPT,), jnp.int32),
            pltpu.VMEM((TPT,), jnp.int32),
            pltpu.VMEM((TPT + MB,), jnp.int32),
            pltpu.VMEM((2, MB), jnp.int32),
            pltpu.VMEM((2, MB), jnp.int32),
            pltpu.VMEM((2, MB, HID), jnp.float32),
            pltpu.VMEM_SHARED((CHUNK_ROWS, HID), jnp.float32),
            pltpu.SemaphoreType.DMA,
            pltpu.SemaphoreType.DMA,
        ],
    )(_scatter_body)
    return f(tok_pad, par_pad, tch)


def _conv_body(tok_hbm, blk_hbm, ts_hbm, c2_hbm, bconv_hbm, out_hbm,
               tokb, blkb, ts_rows, c2_rows, bcv, blk_acc, sem):
    c = lax.axis_index("c")
    s = lax.axis_index("s")
    pltpu.sync_copy(bconv_hbm, bcv)

    zeros16 = jnp.zeros((16,), jnp.float32)

    def zb(r, carry):
        for q in range(8):
            ts_rows[r, pl.ds(q * 16, 16)] = zeros16
        return carry

    lax.fori_loop(0, 128, zb, 0)
    row0 = s * 632
    for j in range(4):
        pltpu.sync_copy(ts_rows, blk_acc.at[pl.ds(row0 + j * 128, 128)])
    pltpu.sync_copy(ts_rows.at[pl.ds(0, 120)],
                    blk_acc.at[pl.ds(row0 + 512, 120)])
    plsc.subcore_barrier()

    tbase = (c * 16 + s) * NPT

    def bat(b, carry):
        nstart = tbase + b * 128
        pltpu.sync_copy(tok_hbm.at[pl.ds(nstart, 128)], tokb)
        pltpu.sync_copy(blk_hbm.at[pl.ds(nstart, 128)], blkb)
        pltpu.async_copy(ts_hbm.at[tokb], ts_rows, sem).wait()
        pltpu.sync_copy(c2_hbm.at[pl.ds(nstart, 128)], c2_rows)

        def relu_b(r, carry2):
            for q in range(8):
                sl = pl.ds(q * 16, 16)
                ts_rows[r, sl] = jnp.maximum(
                    ts_rows[r, sl] + c2_rows[r, sl] + bcv[sl], 0.0)
            return carry2

        lax.fori_loop(0, 128, relu_b, 0)
        pltpu.sync_copy(ts_rows, blk_acc.at[blkb], add=True)
        return carry

    lax.fori_loop(0, NBAT, bat, 0)
    plsc.subcore_barrier()

    for j in range(4):
        pltpu.sync_copy(blk_acc.at[pl.ds(row0 + j * 128, 128)],
                        out_hbm.at[c].at[pl.ds(row0 + j * 128, 128)])
    pltpu.sync_copy(blk_acc.at[pl.ds(row0 + 512, 120)],
                    out_hbm.at[c].at[pl.ds(row0 + 512, 120)])


def _conv(tok_pad, blk_pad, ts, c2, b_conv):
    f = functools.partial(
        pl.kernel,
        out_type=jax.ShapeDtypeStruct((2, BLK_ROWS, HID), jnp.float32),
        mesh=_mesh(),
        compiler_params=_SC_PARAMS,
        scratch_types=[
            pltpu.VMEM((128,), jnp.int32),
            pltpu.VMEM((128,), jnp.int32),
            pltpu.VMEM((128, HID), jnp.float32),
            pltpu.VMEM((128, HID), jnp.float32),
            pltpu.VMEM((HID,), jnp.float32),
            pltpu.VMEM_SHARED((BLK_ROWS, HID), jnp.float32),
            pltpu.SemaphoreType.DMA,
        ],
    )(_conv_body)
    return f(tok_pad, blk_pad, ts, c2, b_conv)


def _block_body(acc_ref, wb_ref, bb_ref, wgi_ref, wgo_ref,
                asgi_ref, adgi_ref, asgo_ref, adgo_ref,
                bf_ref, hgi_aug_ref, hgo_aug_ref,
                o_asgi, o_adgi, o_asgo, o_adgo):
    bs = acc_ref[0] + acc_ref[1]
    bf = jnp.maximum(jnp.dot(bs, wb_ref[...], precision=PREC) + bb_ref[...],
                     0.0)
    bf_ref[...] = bf
    hgi = jnp.dot(bf, wgi_ref[...], precision=PREC)
    hgo = jnp.dot(bf, wgo_ref[...], precision=PREC)
    z = jnp.zeros((bf.shape[0], AUGW - HID // 2), jnp.float32)
    hgi_aug_ref[0] = jnp.concatenate([hgi[:, :64], z], axis=1)
    hgi_aug_ref[1] = jnp.concatenate([hgi[:, 64:], z], axis=1)
    hgo_aug_ref[0] = jnp.concatenate([hgo[:, :64], z], axis=1)
    hgo_aug_ref[1] = jnp.concatenate([hgo[:, 64:], z], axis=1)
    for out_ref, h, a_ref in ((o_asgi, hgi, asgi_ref), (o_adgi, hgi, adgi_ref),
                              (o_asgo, hgo, asgo_ref), (o_adgo, hgo, adgo_ref)):
        out_ref[0] = jnp.dot(h, a_ref[:, 0:2], precision=PREC)
        out_ref[1] = jnp.dot(h, a_ref[:, 2:4], precision=PREC)


def _block(acc, w_block, b_block, w_gi, w_go, a_sgi, a_dgi, a_sgo, a_dgo):
    R = 1000
    grid = (N_BLK // R,)
    full128 = pl.BlockSpec((HID, HID), lambda i: (0, 0))
    alph = pl.BlockSpec((HID, HEADS), lambda i: (0, 0))
    alph_out = pl.BlockSpec((2, R, 2), lambda i: (0, i, 0))
    aug_out = pl.BlockSpec((2, R, AUGW), lambda i: (0, i, 0))
    return pl.pallas_call(
        _block_body,
        grid=grid,
        in_specs=[
            pl.BlockSpec((2, R, HID), lambda i: (0, i, 0)),
            full128,
            pl.BlockSpec((1, HID), lambda i: (0, 0)),
            full128, full128,
            alph, alph, alph, alph,
        ],
        out_specs=[
            pl.BlockSpec((R, HID), lambda i: (i, 0)),
            aug_out, aug_out,
            alph_out, alph_out, alph_out, alph_out,
        ],
        out_shape=(
            jax.ShapeDtypeStruct((N_BLK, HID), jnp.float32),
            jax.ShapeDtypeStruct((2, N_BLK, AUGW), jnp.float32),
            jax.ShapeDtypeStruct((2, N_BLK, AUGW), jnp.float32),
            jax.ShapeDtypeStruct((2, N_BLK, 2), jnp.float32),
            jax.ShapeDtypeStruct((2, N_BLK, 2), jnp.float32),
            jax.ShapeDtypeStruct((2, N_BLK, 2), jnp.float32),
            jax.ShapeDtypeStruct((2, N_BLK, 2), jnp.float32),
        ),
    )(acc, w_block, b_block, w_gi, w_go, a_sgi, a_dgi, a_sgo, a_dgo)


def _gat_body(src_hbm, dst_hbm, et_hbm, asrc_hbm, adst_hbm, etb_hbm, haug_hbm,
              out_hbm, asrc_res, adst_res, etb_res, sb2, db2, tb2, sidx2,
              wb02, wb12, rows2, out_acc, sem0, sem1):
    c = lax.axis_index("c")
    s = lax.axis_index("s")
    pltpu.sync_copy(asrc_hbm.at[c], asrc_res)
    pltpu.sync_copy(adst_hbm.at[c], adst_res)
    pltpu.sync_copy(etb_hbm.at[c], etb_res)
    himask = jnp.full((16,), -65536, jnp.int32)

    zeros16 = jnp.zeros((16,), jnp.float32)

    def zb(r, carry):
        for q in range(AUGW // 16):
            rows2[0, r, pl.ds(q * 16, 16)] = zeros16
        return carry

    lax.fori_loop(0, EB, zb, 0)
    row0 = s * 632
    for j in range(8):
        pltpu.sync_copy(rows2.at[0].at[pl.ds(0, 79)],
                        out_acc.at[pl.ds(row0 + j * 79, 79)])
    plsc.subcore_barrier()

    coff = c * N_BLK
    lane = lax.iota(jnp.int32, 16)
    sems = (sem0, sem1)

    def stage(b, i):
        ebase = s * EPT + b * EB
        pltpu.sync_copy(src_hbm.at[pl.ds(ebase, EB)], sb2.at[i])
        pltpu.sync_copy(dst_hbm.at[pl.ds(ebase, EB)], db2.at[i])
        pltpu.sync_copy(et_hbm.at[pl.ds(ebase, EB)], tb2.at[i])
        for k in range(EB // 16):
            sl = pl.ds(k * 16, 16)
            sv = sb2[i, sl]
            dv = db2[i, sl]
            tv = tb2[i, sl]
            p1 = plsc.load_gather(asrc_res, [sv])
            p2 = plsc.load_gather(adst_res, [dv])
            p3 = plsc.load_gather(etb_res, [tv])
            for h in range(2):
                if h == 0:
                    a1 = plsc.bitcast(jnp.bitwise_and(p1, himask), jnp.float32)
                    a2 = plsc.bitcast(jnp.bitwise_and(p2, himask), jnp.float32)
                    a3 = plsc.bitcast(jnp.bitwise_and(p3, himask), jnp.float32)
                else:
                    a1 = plsc.bitcast(lax.shift_left(p1, 16), jnp.float32)
                    a2 = plsc.bitcast(lax.shift_left(p2, 16), jnp.float32)
                    a3 = plsc.bitcast(lax.shift_left(p3, 16), jnp.float32)
                e = a1 + a2 + a3
                e = jnp.where(e >= 0.0, e, 0.2 * e)
                w = jnp.exp(e)
                if h == 0:
                    wb02[i, sl] = w
                else:
                    wb12[i, sl] = w
            sidx2[i, sl] = sv + coff
        pltpu.async_copy(haug_hbm.at[sidx2.at[i]], rows2.at[i], sems[i])

    def drain(i):
        pltpu.make_async_copy(haug_hbm.at[sidx2.at[i]], rows2.at[i],
                              sems[i]).wait()

        def scale(k, carry2):
            wv0 = wb02[i, pl.ds(k * 16, 16)]
            wv1 = wb12[i, pl.ds(k * 16, 16)]
            for j16 in range(16):
                j = k * 16 + j16
                w0 = wv0[j16]
                w1 = wv1[j16]
                for q in range(2):
                    sl = pl.ds(q * 16, 16)
                    rows2[i, j, sl] = rows2[i, j, sl] * w0
                for q in range(2, 4):
                    sl = pl.ds(q * 16, 16)
                    rows2[i, j, sl] = rows2[i, j, sl] * w1
                rows2[i, j, pl.ds(64, 16)] = jnp.where(
                    lane == 0, w0, jnp.where(lane == 1, w1, 0.0))
            return carry2

        lax.fori_loop(0, EB // 16, scale, 0)
        pltpu.sync_copy(rows2.at[i], out_acc.at[db2.at[i]], add=True)

    stage(0, 0)
    NPAIR = NEB // 2

    def bat2(p, carry):
        stage(2 * p + 1, 1)
        drain(0)

        @pl.when(p < NPAIR - 1)
        def _():
            stage(2 * p + 2, 0)

        drain(1)
        return carry

    lax.fori_loop(0, NPAIR, bat2, 0)
    plsc.subcore_barrier()

    for j in range(4):
        pltpu.sync_copy(out_acc.at[pl.ds(row0 + j * 128, 128)],
                        out_hbm.at[c].at[pl.ds(row0 + j * 128, 128)])
    pltpu.sync_copy(out_acc.at[pl.ds(row0 + 512, 120)],
                    out_hbm.at[c].at[pl.ds(row0 + 512, 120)])


def _gat(srcx, dstx, etx, asrc_sc, adst_sc, etb_sc, haug):
    f = functools.partial(
        pl.kernel,
        out_type=jax.ShapeDtypeStruct((2, BLK_ROWS, AUGW), jnp.float32),
        mesh=_mesh(),
        compiler_params=_SC_PARAMS,
        scratch_types=[
            pltpu.VMEM((N_BLK,), jnp.int32),
            pltpu.VMEM((N_BLK,), jnp.int32),
            pltpu.VMEM((8,), jnp.int32),
            pltpu.VMEM((2, EB), jnp.int32),
            pltpu.VMEM((2, EB), jnp.int32),
            pltpu.VMEM((2, EB), jnp.int32),
            pltpu.VMEM((2, EB), jnp.int32),
            pltpu.VMEM((2, EB), jnp.float32),
            pltpu.VMEM((2, EB), jnp.float32),
            pltpu.VMEM((2, EB, AUGW), jnp.float32),
            pltpu.VMEM_SHARED((BLK_ROWS, AUGW), jnp.float32),
            pltpu.SemaphoreType.DMA,
            pltpu.SemaphoreType.DMA,
        ],
    )(_gat_body)
    return f(srcx, dstx, etx, asrc_sc, adst_sc, etb_sc, haug)


def _final_body(bf_ref, ain_ref, aout_ref, wm_ref, bm_ref, lab_ref,
                sig_ref, loss_ref, pooled_ref, macc):
    i = pl.program_id(0)

    @pl.when(i == 0)
    def _():
        macc[...] = jnp.full((1, HID), -jnp.inf, jnp.float32)

    parts = []
    for c in range(2):
        for g in range(2):
            num_i = ain_ref[c, :, g * DH:(g + 1) * DH]
            den_i = ain_ref[c, :, 64 + g:65 + g]
            num_o = aout_ref[c, :, g * DH:(g + 1) * DH]
            den_o = aout_ref[c, :, 64 + g:65 + g]
            parts.append(num_i / (den_i + 1e-9) + num_o / (den_o + 1e-9))
    fea = bf_ref[...] + jnp.concatenate(parts, axis=1)
    macc[...] = jnp.maximum(macc[...], jnp.max(fea, axis=0, keepdims=True))

    @pl.when(i == pl.num_programs(0) - 1)
    def _():
        pooled = macc[...]
        logits = jnp.dot(pooled, wm_ref[...], precision=PREC) + bm_ref[...]
        pooled_ref[...] = pooled
        sig_ref[...] = 1.0 / (1.0 + jnp.exp(-logits))
        l = logits[0, 0]
        y = lab_ref[0, 0]
        loss_ref[...] = (jnp.maximum(l, 0.0) - l * y +
                         jnp.log1p(jnp.exp(-jnp.abs(l)))).reshape(1, 1)


def _final(bf, accin, accout, w_mlp, b_mlp, lab):
    R = 1000
    grid = (N_BLK // R,)
    return pl.pallas_call(
        _final_body,
        grid=grid,
        in_specs=[
            pl.BlockSpec((R, HID), lambda i: (i, 0)),
            pl.BlockSpec((2, R, AUGW), lambda i: (0, i, 0)),
            pl.BlockSpec((2, R, AUGW), lambda i: (0, i, 0)),
            pl.BlockSpec((HID, 1), lambda i: (0, 0)),
            pl.BlockSpec((1, 1), lambda i: (0, 0)),
            pl.BlockSpec((1, 1), lambda i: (0, 0)),
        ],
        out_specs=[
            pl.BlockSpec((1, 1), lambda i: (0, 0)),
            pl.BlockSpec((1, 1), lambda i: (0, 0)),
            pl.BlockSpec((1, HID), lambda i: (0, 0)),
        ],
        out_shape=(
            jax.ShapeDtypeStruct((1, 1), jnp.float32),
            jax.ShapeDtypeStruct((1, 1), jnp.float32),
            jax.ShapeDtypeStruct((1, HID), jnp.float32),
        ),
        scratch_shapes=[pltpu.VMEM((1, HID), jnp.float32)],
    )(bf, accin, accout, w_mlp, b_mlp, lab)


def _pack_pair(a):
    bb = lax.bitcast_convert_type(a.astype(jnp.bfloat16),
                                  jnp.uint16).astype(jnp.uint32)
    return (jnp.left_shift(bb[..., 0], 16) | bb[..., 1]).astype(jnp.int32)


def _alpha_mat(a):
    m = jnp.zeros((HID, HEADS), jnp.float32)
    for h in range(HEADS):
        m = m.at[h * DH:(h + 1) * DH, h].set(a[h])
    return m


def kernel(ast_tokens, ast_parent, emb_block_id, cfg_edge_index, cfg_edge_type,
           label, embed_table, W_self, W_child, b_conv, W_block, b_block,
           W_gi, a_src_gi, a_dst_gi, et_gi,
           W_go, a_src_go, a_dst_go, et_go,
           W_mlp, b_mlp):
    i32 = jnp.int32
    npad = PAD_N - N_AST
    tok_pad = jnp.concatenate([ast_tokens.astype(i32),
                               jnp.zeros((npad,), i32)])
    par_pad = jnp.concatenate([ast_parent.astype(i32),
                               jnp.full((npad,), -1, i32)])
    blk_pad = jnp.concatenate([emb_block_id.astype(i32),
                               jnp.full((npad,), N_BLK, i32)])

    ts, tch = _prep(embed_table, W_self, W_child)
    c2 = _scatter(tok_pad, par_pad, tch)
    acc = _conv(tok_pad, blk_pad, ts, c2, b_conv)

    bf, haug_gi, haug_go, asgi, adgi, asgo, adgo = _block(
        acc, W_block, b_block.reshape(1, HID), W_gi, W_go,
        _alpha_mat(a_src_gi), _alpha_mat(a_dst_gi),
        _alpha_mat(a_src_go), _alpha_mat(a_dst_go))

    srcx = cfg_edge_index[0].astype(i32)
    dstx = cfg_edge_index[1].astype(i32)
    etx = cfg_edge_type.astype(i32)
    asgi = _pack_pair(asgi)
    adgi = _pack_pair(adgi)
    asgo = _pack_pair(asgo)
    adgo = _pack_pair(adgo)
    etb_gi = _pack_pair(jnp.pad(
        et_gi.reshape(3, 2, 2).transpose(1, 0, 2), ((0, 0), (0, 5), (0, 0))))
    etb_go = _pack_pair(jnp.pad(
        et_go.reshape(3, 2, 2).transpose(1, 0, 2), ((0, 0), (0, 5), (0, 0))))

    accin = _gat(srcx, dstx, etx, asgi, adgi, etb_gi,
                 haug_gi.reshape(2 * N_BLK, AUGW))
    accout = _gat(dstx, srcx, etx, asgo, adgo, etb_go,
                  haug_go.reshape(2 * N_BLK, AUGW))

    sig, loss, pooled = _final(bf, accin, accout, W_mlp,
                               b_mlp.reshape(1, 1), label.reshape(1, 1))
    return (sig, loss.reshape(()), pooled)
